# Initial kernel scaffold; baseline (speedup 1.0000x reference)
import jax
import jax.numpy as jnp
from jax.experimental import pallas as pl


def kernel(x, edge_index, batch, W1, b1, W2, b2, Wlin, blin):
    raise NotImplementedError("write your pallas kernel here")


# speedup vs baseline: 9.5599x; 9.5599x over previous
import functools

import jax
import jax.numpy as jnp
from jax import lax
from jax.experimental import pallas as pl
from jax.experimental.pallas import tpu as pltpu, tpu_sc as plsc

N = 50000
E = 800000
G = 64
F_HID = 128
F_OUT = 64

NP = 51200
EP = 802816
NC = 2
NS = 16
ZR = NP // NS
K = 128

EPT2 = EP // (NC * NS)
NB2 = EPT2 // K
EPT1 = EP // NS
NB1 = EPT1 // K

BN = 512
NBLK = NP // BN

_mesh = plsc.VectorSubcoreMesh(core_axis_name="c", subcore_axis_name="s")
_f32 = jnp.float32
_sc_params = pltpu.CompilerParams(use_tc_tiling_on_sc=False)


@functools.partial(
    pl.kernel,
    out_type=jax.ShapeDtypeStruct((NC, NP, 8), _f32),
    compiler_params=_sc_params,
    mesh=_mesh,
    scratch_types=[
        pltpu.VMEM((K,), jnp.int32),
        pltpu.VMEM((K, 8), _f32),
        pltpu.VMEM_SHARED((NP, 8), _f32),
        pltpu.SemaphoreType.DMA,
    ],
)
def _deg_kernel(dst_hbm, ones_hbm, zeros_hbm, out_hbm, dstv, onesv, acc, sem):
    cid = lax.axis_index("c")
    sid = lax.axis_index("s")
    base = (cid * NS + sid) * EPT2
    pltpu.sync_copy(zeros_hbm, acc.at[pl.ds(sid * ZR, ZR)])
    pltpu.sync_copy(ones_hbm, onesv)
    plsc.subcore_barrier()

    def body(b, carry):
        pltpu.sync_copy(dst_hbm.at[pl.ds(base + b * K, K)], dstv)
        pltpu.sync_copy(onesv, acc.at[dstv], add=True)
        return carry

    lax.fori_loop(0, NB2, body, 0)
    plsc.subcore_barrier()
    pltpu.sync_copy(acc.at[pl.ds(sid * ZR, ZR)],
                    out_hbm.at[cid, pl.ds(sid * ZR, ZR)])


@functools.partial(
    pl.kernel,
    out_type=jax.ShapeDtypeStruct((NC, NP, 16), _f32),
    compiler_params=_sc_params,
    mesh=_mesh,
    scratch_types=[
        pltpu.VMEM((K,), jnp.int32),
        pltpu.VMEM((K,), jnp.int32),
        pltpu.VMEM((K, 16), _f32),
        pltpu.VMEM_SHARED((NP, 16), _f32),
        pltpu.SemaphoreType.DMA,
    ],
)
def _prop1_kernel(src_hbm, dst_hbm, xp_hbm, zeros_hbm, out_hbm,
                  srcv, dstv, rows, acc, sem):
    cid = lax.axis_index("c")
    sid = lax.axis_index("s")
    base = (cid * NS + sid) * EPT2
    pltpu.sync_copy(zeros_hbm, acc.at[pl.ds(sid * ZR, ZR)])
    plsc.subcore_barrier()

    def body(b, carry):
        pltpu.sync_copy(src_hbm.at[pl.ds(base + b * K, K)], srcv)
        pltpu.sync_copy(dst_hbm.at[pl.ds(base + b * K, K)], dstv)
        pltpu.async_copy(xp_hbm.at[srcv], rows, sem).wait()
        pltpu.sync_copy(rows, acc.at[dstv], add=True)
        return carry

    lax.fori_loop(0, NB2, body, 0)
    plsc.subcore_barrier()
    pltpu.sync_copy(acc.at[pl.ds(sid * ZR, ZR)],
                    out_hbm.at[cid, pl.ds(sid * ZR, ZR)])


@functools.partial(
    pl.kernel,
    out_type=[jax.ShapeDtypeStruct((NP, 32), _f32)] * 4,
    compiler_params=_sc_params,
    mesh=_mesh,
    scratch_types=[
        pltpu.VMEM((K,), jnp.int32),
        pltpu.VMEM((K,), jnp.int32),
        pltpu.VMEM((K, 32), _f32),
        pltpu.VMEM_SHARED((NP, 32), _f32),
        pltpu.SemaphoreType.DMA,
    ],
)
def _prop2_kernel(src_hbm, dst_hbm, h0_hbm, h1_hbm, h2_hbm, h3_hbm, zeros_hbm,
                  o0, o1, o2, o3, srcv, dstv, rows, acc, sem):
    cid = lax.axis_index("c")
    sid = lax.axis_index("s")
    base = sid * EPT1

    def one_group(tbl_hbm, out_hbm):
        pltpu.sync_copy(zeros_hbm, acc.at[pl.ds(sid * ZR, ZR)])
        plsc.subcore_barrier()

        def body(b, carry):
            pltpu.sync_copy(src_hbm.at[pl.ds(base + b * K, K)], srcv)
            pltpu.sync_copy(dst_hbm.at[pl.ds(base + b * K, K)], dstv)
            pltpu.async_copy(tbl_hbm.at[srcv], rows, sem).wait()
            pltpu.sync_copy(rows, acc.at[dstv], add=True)
            return carry

        lax.fori_loop(0, NB1, body, 0)
        plsc.subcore_barrier()
        pltpu.sync_copy(acc.at[pl.ds(sid * ZR, ZR)],
                        out_hbm.at[pl.ds(sid * ZR, ZR)])
        plsc.subcore_barrier()

    @pl.when(cid == 0)
    def _():
        one_group(h0_hbm, o0)
        one_group(h1_hbm, o1)

    @pl.when(cid == 1)
    def _():
        one_group(h2_hbm, o2)
        one_group(h3_hbm, o3)


def _prep_body(degp_ref, x16_ref, xp_ref, dinv_ref):
    deg = degp_ref[0, :, 0:1] + degp_ref[1, :, 0:1] + 1.0
    dinv = lax.rsqrt(deg)
    dinv_ref[...] = dinv
    xp_ref[...] = x16_ref[...] * dinv


def _prep_call(degp, x16):
    return pl.pallas_call(
        _prep_body,
        grid=(NBLK,),
        in_specs=[
            pl.BlockSpec((NC, BN, 8), lambda i: (0, i, 0)),
            pl.BlockSpec((BN, 16), lambda i: (i, 0)),
        ],
        out_specs=[
            pl.BlockSpec((BN, 16), lambda i: (i, 0)),
            pl.BlockSpec((BN, 1), lambda i: (i, 0)),
        ],
        out_shape=[
            jax.ShapeDtypeStruct((NP, 16), _f32),
            jax.ShapeDtypeStruct((NP, 1), _f32),
        ],
    )(degp, x16)


def _h1_body(p1p_ref, xp_ref, dinv_ref, w1_ref, b1_ref, o0, o1, o2, o3):
    dinv = dinv_ref[...]
    p = dinv * (p1p_ref[0] + p1p_ref[1] + xp_ref[...])
    h = jnp.dot(p, w1_ref[...], preferred_element_type=_f32) + b1_ref[...]
    hp = dinv * jnp.maximum(h, 0.0)
    o0[...] = hp[:, 0:32]
    o1[...] = hp[:, 32:64]
    o2[...] = hp[:, 64:96]
    o3[...] = hp[:, 96:128]


def _h1_call(p1p, xp, dinv, w1p, b1r):
    return pl.pallas_call(
        _h1_body,
        grid=(NBLK,),
        in_specs=[
            pl.BlockSpec((NC, BN, 16), lambda i: (0, i, 0)),
            pl.BlockSpec((BN, 16), lambda i: (i, 0)),
            pl.BlockSpec((BN, 1), lambda i: (i, 0)),
            pl.BlockSpec((16, F_HID), lambda i: (0, 0)),
            pl.BlockSpec((1, F_HID), lambda i: (0, 0)),
        ],
        out_specs=[pl.BlockSpec((BN, 32), lambda i: (i, 0))] * 4,
        out_shape=[jax.ShapeDtypeStruct((NP, 32), _f32)] * 4,
    )(p1p, xp, dinv, w1p, b1r)


def _fin_body(p20, p21, p22, p23, h0, h1, h2, h3, dinv_ref,
              w2_ref, b2_ref, wl_ref, bl_ref, bf_ref, out_ref, acc):
    i = pl.program_id(0)

    @pl.when(i == 0)
    def _():
        acc[...] = jnp.zeros((G, F_HID), _f32)

    dinv = dinv_ref[...]
    p2 = jnp.concatenate([p20[...], p21[...], p22[...], p23[...]], axis=1)
    hp = jnp.concatenate([h0[...], h1[...], h2[...], h3[...]], axis=1)
    pf = dinv * (p2 + hp)
    h2v = jnp.dot(pf, w2_ref[...], preferred_element_type=_f32) + b2_ref[...]
    h2v = jnp.maximum(h2v, 0.0)
    h3v = lax.dot_general(h2v, wl_ref[...], (((1,), (1,)), ((), ())),
                          preferred_element_type=_f32) + bl_ref[...]
    bf = bf_ref[...]
    gids = lax.broadcasted_iota(jnp.int32, (BN, G), 1).astype(_f32)
    oh = (bf == gids).astype(_f32)
    ext = jnp.concatenate(
        [h3v, jnp.ones((BN, 1), _f32), jnp.zeros((BN, F_HID - F_OUT - 1), _f32)],
        axis=1)
    acc[...] += lax.dot_general(oh, ext, (((0,), (0,)), ((), ())),
                                preferred_element_type=_f32)

    @pl.when(i == NBLK - 1)
    def _():
        out_ref[...] = acc[:, 0:F_OUT] / jnp.maximum(acc[:, F_OUT:F_OUT + 1], 1.0)


def _fin_call(p2s, hps, dinv, w2, b2r, wl, blr, bf):
    nblock = pl.BlockSpec((BN, 32), lambda i: (i, 0))
    return pl.pallas_call(
        _fin_body,
        grid=(NBLK,),
        in_specs=[nblock] * 8 + [
            pl.BlockSpec((BN, 1), lambda i: (i, 0)),
            pl.BlockSpec((F_HID, F_HID), lambda i: (0, 0)),
            pl.BlockSpec((1, F_HID), lambda i: (0, 0)),
            pl.BlockSpec((F_OUT, F_HID), lambda i: (0, 0)),
            pl.BlockSpec((1, F_OUT), lambda i: (0, 0)),
            pl.BlockSpec((BN, 1), lambda i: (i, 0)),
        ],
        out_specs=pl.BlockSpec((G, F_OUT), lambda i: (0, 0)),
        out_shape=jax.ShapeDtypeStruct((G, F_OUT), _f32),
        scratch_shapes=[pltpu.VMEM((G, F_HID), _f32)],
    )(*p2s, *hps, dinv, w2, b2r, wl, blr, bf)


def kernel(x, edge_index, batch, W1, b1, W2, b2, Wlin, blin):
    src = edge_index[0]
    dst = edge_index[1]
    pad_e = EP - E
    src_p = jnp.concatenate([src, jnp.zeros((pad_e,), jnp.int32)])
    dst_p = jnp.concatenate([dst, jnp.full((pad_e,), N, jnp.int32)])

    x16 = jnp.zeros((NP, 16), _f32).at[:N, :7].set(x)
    w1p = jnp.zeros((16, F_HID), _f32).at[:7, :].set(W1)
    b1r = b1.reshape(1, F_HID)
    b2r = b2.reshape(1, F_HID)
    blr = blin.reshape(1, F_OUT)
    bf = jnp.full((NP, 1), float(G), _f32).at[:N, 0].set(batch.astype(_f32))

    ones8 = jnp.ones((K, 8), _f32)
    zeros8 = jnp.zeros((ZR, 8), _f32)
    zeros16 = jnp.zeros((ZR, 16), _f32)
    zeros32 = jnp.zeros((ZR, 32), _f32)

    degp = _deg_kernel(dst_p, ones8, zeros8)
    xp, dinv = _prep_call(degp, x16)
    p1p = _prop1_kernel(src_p, dst_p, xp, zeros16)
    hps = _h1_call(p1p, xp, dinv, w1p, b1r)
    p2s = _prop2_kernel(src_p, dst_p, *hps, zeros32)
    return _fin_call(p2s, hps, dinv, W2, b2r, Wlin, blr, bf)


# speedup vs baseline: 13.6627x; 1.4292x over previous
import functools

import jax
import jax.numpy as jnp
from jax import lax
from jax.experimental import pallas as pl
from jax.experimental.pallas import tpu as pltpu, tpu_sc as plsc

N = 50000
E = 800000
G = 64
F_HID = 128
F_OUT = 64

NP = 51200
K = 128
EP = 819200
EPR = EP // K
NC = 2
NS = 16
ZR = NP // NS

RT2 = EPR // (NC * NS)
C3 = 8
NIT2 = RT2 // C3
RT1 = EPR // NS
C5 = 5
NIT1 = RT1 // C5

BN = 512
NBLK = NP // BN

_mesh = plsc.VectorSubcoreMesh(core_axis_name="c", subcore_axis_name="s")
_f32 = jnp.float32
_sc_params = pltpu.CompilerParams(use_tc_tiling_on_sc=False)


@functools.partial(
    pl.kernel,
    out_type=jax.ShapeDtypeStruct((NC, NP, 8), _f32),
    compiler_params=_sc_params,
    mesh=_mesh,
    scratch_types=[
        pltpu.VMEM((C3, K), jnp.int32),
        pltpu.VMEM((K, 8), _f32),
        pltpu.VMEM_SHARED((NP, 8), _f32),
        pltpu.SemaphoreType.DMA,
    ],
)
def _deg_kernel(dst2_hbm, ones_hbm, zeros_hbm, out_hbm, dstb, onesv, acc, ss):
    cid = lax.axis_index("c")
    sid = lax.axis_index("s")
    base = (cid * NS + sid) * RT2
    pltpu.sync_copy(zeros_hbm, acc.at[pl.ds(sid * ZR, ZR)])
    pltpu.sync_copy(ones_hbm, onesv)
    plsc.subcore_barrier()

    def it(t, carry):
        @pl.when(t > 0)
        def _():
            for _j in range(C3):
                pltpu.make_async_copy(ones_hbm, onesv, ss).wait()

        pltpu.sync_copy(dst2_hbm.at[pl.ds(base + t * C3, C3)], dstb)
        for j in range(C3):
            pltpu.async_copy(onesv, acc.at[dstb.at[j]], ss, add=True)
        return carry

    lax.fori_loop(0, NIT2, it, 0)
    for _j in range(C3):
        pltpu.make_async_copy(ones_hbm, onesv, ss).wait()
    plsc.subcore_barrier()
    pltpu.sync_copy(acc.at[pl.ds(sid * ZR, ZR)],
                    out_hbm.at[cid, pl.ds(sid * ZR, ZR)])


@functools.partial(
    pl.kernel,
    out_type=jax.ShapeDtypeStruct((NC, NP, 16), _f32),
    compiler_params=_sc_params,
    mesh=_mesh,
    scratch_types=[
        pltpu.VMEM((C3, K), jnp.int32),
        pltpu.VMEM((C3, K), jnp.int32),
        pltpu.VMEM((C3 * K, 16), _f32),
        pltpu.VMEM_SHARED((NP, 16), _f32),
        pltpu.SemaphoreType.DMA,
        pltpu.SemaphoreType.DMA,
    ],
)
def _prop1_kernel(src2_hbm, dst2_hbm, xp_hbm, zeros_hbm, out_hbm,
                  srcb, dstb, rows, acc, sg, ss):
    cid = lax.axis_index("c")
    sid = lax.axis_index("s")
    base = (cid * NS + sid) * RT2
    dummy = out_hbm.at[0, pl.ds(0, C3 * K)]
    pltpu.sync_copy(zeros_hbm, acc.at[pl.ds(sid * ZR, ZR)])
    plsc.subcore_barrier()

    def it(t, carry):
        @pl.when(t > 0)
        def _():
            pltpu.make_async_copy(dummy, rows, ss).wait()

        pltpu.sync_copy(src2_hbm.at[pl.ds(base + t * C3, C3)], srcb)
        pltpu.sync_copy(dst2_hbm.at[pl.ds(base + t * C3, C3)], dstb)
        gs = [pltpu.async_copy(xp_hbm.at[srcb.at[j]],
                               rows.at[pl.ds(j * K, K)], sg)
              for j in range(C3)]
        for j in range(C3):
            gs[j].wait()
            pltpu.async_copy(rows.at[pl.ds(j * K, K)],
                             acc.at[dstb.at[j]], ss, add=True)
        return carry

    lax.fori_loop(0, NIT2, it, 0)
    pltpu.make_async_copy(dummy, rows, ss).wait()
    plsc.subcore_barrier()
    pltpu.sync_copy(acc.at[pl.ds(sid * ZR, ZR)],
                    out_hbm.at[cid, pl.ds(sid * ZR, ZR)])


@functools.partial(
    pl.kernel,
    out_type=[jax.ShapeDtypeStruct((NP, 32), _f32)] * 4,
    compiler_params=_sc_params,
    mesh=_mesh,
    scratch_types=[
        pltpu.VMEM((C5, K), jnp.int32),
        pltpu.VMEM((C5, K), jnp.int32),
        pltpu.VMEM((C5 * K, 32), _f32),
        pltpu.VMEM_SHARED((NP, 32), _f32),
        pltpu.SemaphoreType.DMA,
        pltpu.SemaphoreType.DMA,
    ],
)
def _prop2_kernel(src2_hbm, dst2_hbm, h0_hbm, h1_hbm, h2_hbm, h3_hbm, zeros_hbm,
                  o0, o1, o2, o3, srcb, dstb, rows, acc, sg, ss):
    cid = lax.axis_index("c")
    sid = lax.axis_index("s")
    base = sid * RT1
    dummy = o0.at[pl.ds(0, C5 * K)]

    def one_group(tbl_hbm, out_hbm):
        pltpu.sync_copy(zeros_hbm, acc.at[pl.ds(sid * ZR, ZR)])
        plsc.subcore_barrier()

        def it(t, carry):
            @pl.when(t > 0)
            def _():
                pltpu.make_async_copy(dummy, rows, ss).wait()

            pltpu.sync_copy(src2_hbm.at[pl.ds(base + t * C5, C5)], srcb)
            pltpu.sync_copy(dst2_hbm.at[pl.ds(base + t * C5, C5)], dstb)
            gs = [pltpu.async_copy(tbl_hbm.at[srcb.at[j]],
                                   rows.at[pl.ds(j * K, K)], sg)
                  for j in range(C5)]
            for j in range(C5):
                gs[j].wait()
                pltpu.async_copy(rows.at[pl.ds(j * K, K)],
                                 acc.at[dstb.at[j]], ss, add=True)
            return carry

        lax.fori_loop(0, NIT1, it, 0)
        pltpu.make_async_copy(dummy, rows, ss).wait()
        plsc.subcore_barrier()
        pltpu.sync_copy(acc.at[pl.ds(sid * ZR, ZR)],
                        out_hbm.at[pl.ds(sid * ZR, ZR)])
        plsc.subcore_barrier()

    @pl.when(cid == 0)
    def _():
        one_group(h0_hbm, o0)
        one_group(h1_hbm, o1)

    @pl.when(cid == 1)
    def _():
        one_group(h2_hbm, o2)
        one_group(h3_hbm, o3)


def _prep_body(degp_ref, x16_ref, xp_ref, dinv_ref):
    deg = degp_ref[0, :, 0:1] + degp_ref[1, :, 0:1] + 1.0
    dinv = lax.rsqrt(deg)
    dinv_ref[...] = dinv
    xp_ref[...] = x16_ref[...] * dinv


def _prep_call(degp, x16):
    return pl.pallas_call(
        _prep_body,
        grid=(NBLK,),
        in_specs=[
            pl.BlockSpec((NC, BN, 8), lambda i: (0, i, 0)),
            pl.BlockSpec((BN, 16), lambda i: (i, 0)),
        ],
        out_specs=[
            pl.BlockSpec((BN, 16), lambda i: (i, 0)),
            pl.BlockSpec((BN, 1), lambda i: (i, 0)),
        ],
        out_shape=[
            jax.ShapeDtypeStruct((NP, 16), _f32),
            jax.ShapeDtypeStruct((NP, 1), _f32),
        ],
    )(degp, x16)


def _h1_body(p1p_ref, xp_ref, dinv_ref, w1_ref, b1_ref, o0, o1, o2, o3):
    dinv = dinv_ref[...]
    p = dinv * (p1p_ref[0] + p1p_ref[1] + xp_ref[...])
    h = jnp.dot(p, w1_ref[...], preferred_element_type=_f32) + b1_ref[...]
    hp = dinv * jnp.maximum(h, 0.0)
    o0[...] = hp[:, 0:32]
    o1[...] = hp[:, 32:64]
    o2[...] = hp[:, 64:96]
    o3[...] = hp[:, 96:128]


def _h1_call(p1p, xp, dinv, w1p, b1r):
    return pl.pallas_call(
        _h1_body,
        grid=(NBLK,),
        in_specs=[
            pl.BlockSpec((NC, BN, 16), lambda i: (0, i, 0)),
            pl.BlockSpec((BN, 16), lambda i: (i, 0)),
            pl.BlockSpec((BN, 1), lambda i: (i, 0)),
            pl.BlockSpec((16, F_HID), lambda i: (0, 0)),
            pl.BlockSpec((1, F_HID), lambda i: (0, 0)),
        ],
        out_specs=[pl.BlockSpec((BN, 32), lambda i: (i, 0))] * 4,
        out_shape=[jax.ShapeDtypeStruct((NP, 32), _f32)] * 4,
    )(p1p, xp, dinv, w1p, b1r)


def _fin_body(p20, p21, p22, p23, h0, h1, h2, h3, dinv_ref,
              w2_ref, b2_ref, wl_ref, bl_ref, bf_ref, out_ref, acc):
    i = pl.program_id(0)

    @pl.when(i == 0)
    def _():
        acc[...] = jnp.zeros((G, F_HID), _f32)

    dinv = dinv_ref[...]
    p2 = jnp.concatenate([p20[...], p21[...], p22[...], p23[...]], axis=1)
    hp = jnp.concatenate([h0[...], h1[...], h2[...], h3[...]], axis=1)
    pf = dinv * (p2 + hp)
    h2v = jnp.dot(pf, w2_ref[...], preferred_element_type=_f32) + b2_ref[...]
    h2v = jnp.maximum(h2v, 0.0)
    h3v = lax.dot_general(h2v, wl_ref[...], (((1,), (1,)), ((), ())),
                          preferred_element_type=_f32) + bl_ref[...]
    bf = bf_ref[...]
    gids = lax.broadcasted_iota(jnp.int32, (BN, G), 1).astype(_f32)
    oh = (bf == gids).astype(_f32)
    ext = jnp.concatenate(
        [h3v, jnp.ones((BN, 1), _f32), jnp.zeros((BN, F_HID - F_OUT - 1), _f32)],
        axis=1)
    acc[...] += lax.dot_general(oh, ext, (((0,), (0,)), ((), ())),
                                preferred_element_type=_f32)

    @pl.when(i == NBLK - 1)
    def _():
        out_ref[...] = acc[:, 0:F_OUT] / jnp.maximum(acc[:, F_OUT:F_OUT + 1], 1.0)


def _fin_call(p2s, hps, dinv, w2, b2r, wl, blr, bf):
    nblock = pl.BlockSpec((BN, 32), lambda i: (i, 0))
    return pl.pallas_call(
        _fin_body,
        grid=(NBLK,),
        in_specs=[nblock] * 8 + [
            pl.BlockSpec((BN, 1), lambda i: (i, 0)),
            pl.BlockSpec((F_HID, F_HID), lambda i: (0, 0)),
            pl.BlockSpec((1, F_HID), lambda i: (0, 0)),
            pl.BlockSpec((F_OUT, F_HID), lambda i: (0, 0)),
            pl.BlockSpec((1, F_OUT), lambda i: (0, 0)),
            pl.BlockSpec((BN, 1), lambda i: (i, 0)),
        ],
        out_specs=pl.BlockSpec((G, F_OUT), lambda i: (0, 0)),
        out_shape=jax.ShapeDtypeStruct((G, F_OUT), _f32),
        scratch_shapes=[pltpu.VMEM((G, F_HID), _f32)],
    )(*p2s, *hps, dinv, w2, b2r, wl, blr, bf)


def kernel(x, edge_index, batch, W1, b1, W2, b2, Wlin, blin):
    src = edge_index[0]
    dst = edge_index[1]
    pad_e = EP - E
    src2 = jnp.concatenate([src, jnp.zeros((pad_e,), jnp.int32)]).reshape(EPR, K)
    dst2 = jnp.concatenate([dst, jnp.full((pad_e,), N, jnp.int32)]).reshape(EPR, K)

    x16 = jnp.zeros((NP, 16), _f32).at[:N, :7].set(x)
    w1p = jnp.zeros((16, F_HID), _f32).at[:7, :].set(W1)
    b1r = b1.reshape(1, F_HID)
    b2r = b2.reshape(1, F_HID)
    blr = blin.reshape(1, F_OUT)
    bf = jnp.full((NP, 1), float(G), _f32).at[:N, 0].set(batch.astype(_f32))

    ones8 = jnp.ones((K, 8), _f32)
    zeros8 = jnp.zeros((ZR, 8), _f32)
    zeros16 = jnp.zeros((ZR, 16), _f32)
    zeros32 = jnp.zeros((ZR, 32), _f32)

    degp = _deg_kernel(dst2, ones8, zeros8)
    xp, dinv = _prep_call(degp, x16)
    p1p = _prop1_kernel(src2, dst2, xp, zeros16)
    hps = _h1_call(p1p, xp, dinv, w1p, b1r)
    p2s = _prop2_kernel(src2, dst2, *hps, zeros32)
    return _fin_call(p2s, hps, dinv, W2, b2r, Wlin, blr, bf)


# speedup vs baseline: 15.6426x; 1.1449x over previous
import functools

import jax
import jax.numpy as jnp
from jax import lax
from jax.experimental import pallas as pl
from jax.experimental.pallas import tpu as pltpu, tpu_sc as plsc

N = 50000
E = 800000
G = 64
F_HID = 128
F_OUT = 64

NP = 51200
K = 128
EP = 819200
EPR = EP // K
NC = 2
NS = 16
ZR = NP // NS

RT2 = EPR // (NC * NS)
C3 = 8
NIT2 = RT2 // C3
RT1 = EPR // NS
C5 = 10
NIT1 = RT1 // C5
NG = 8

BN = 512
NBLK = NP // BN

_mesh = plsc.VectorSubcoreMesh(core_axis_name="c", subcore_axis_name="s")
_f32 = jnp.float32
_sc_params = pltpu.CompilerParams(use_tc_tiling_on_sc=False)


@functools.partial(
    pl.kernel,
    out_type=jax.ShapeDtypeStruct((NC, NP, 8), _f32),
    compiler_params=_sc_params,
    mesh=_mesh,
    scratch_types=[
        pltpu.VMEM((C3, K), jnp.int32),
        pltpu.VMEM((K, 8), _f32),
        pltpu.VMEM_SHARED((NP, 8), _f32),
        pltpu.SemaphoreType.DMA,
    ],
)
def _deg_kernel(dst2_hbm, ones_hbm, zeros_hbm, out_hbm, dstb, onesv, acc, ss):
    cid = lax.axis_index("c")
    sid = lax.axis_index("s")
    base = (cid * NS + sid) * RT2
    pltpu.sync_copy(zeros_hbm, acc.at[pl.ds(sid * ZR, ZR)])
    pltpu.sync_copy(ones_hbm, onesv)
    plsc.subcore_barrier()

    def it(t, carry):
        @pl.when(t > 0)
        def _():
            for _j in range(C3):
                pltpu.make_async_copy(ones_hbm, onesv, ss).wait()

        pltpu.sync_copy(dst2_hbm.at[pl.ds(base + t * C3, C3)], dstb)
        for j in range(C3):
            pltpu.async_copy(onesv, acc.at[dstb.at[j]], ss, add=True)
        return carry

    lax.fori_loop(0, NIT2, it, 0)
    for _j in range(C3):
        pltpu.make_async_copy(ones_hbm, onesv, ss).wait()
    plsc.subcore_barrier()
    pltpu.sync_copy(acc.at[pl.ds(sid * ZR, ZR)],
                    out_hbm.at[cid, pl.ds(sid * ZR, ZR)])


@functools.partial(
    pl.kernel,
    out_type=jax.ShapeDtypeStruct((NC, NP, 16), _f32),
    compiler_params=_sc_params,
    mesh=_mesh,
    scratch_types=[
        pltpu.VMEM((C3, K), jnp.int32),
        pltpu.VMEM((C3, K), jnp.int32),
        pltpu.VMEM((C3 * K, 16), _f32),
        pltpu.VMEM_SHARED((NP, 16), _f32),
        pltpu.SemaphoreType.DMA,
        pltpu.SemaphoreType.DMA,
    ],
)
def _prop1_kernel(src2_hbm, dst2_hbm, xp_hbm, zeros_hbm, out_hbm,
                  srcb, dstb, rows, acc, sg, ss):
    cid = lax.axis_index("c")
    sid = lax.axis_index("s")
    base = (cid * NS + sid) * RT2
    dummy = out_hbm.at[0, pl.ds(0, C3 * K)]
    pltpu.sync_copy(zeros_hbm, acc.at[pl.ds(sid * ZR, ZR)])
    plsc.subcore_barrier()

    def it(t, carry):
        @pl.when(t > 0)
        def _():
            pltpu.make_async_copy(dummy, rows, ss).wait()

        pltpu.sync_copy(src2_hbm.at[pl.ds(base + t * C3, C3)], srcb)
        pltpu.sync_copy(dst2_hbm.at[pl.ds(base + t * C3, C3)], dstb)
        gs = [pltpu.async_copy(xp_hbm.at[srcb.at[j]],
                               rows.at[pl.ds(j * K, K)], sg)
              for j in range(C3)]
        for j in range(C3):
            gs[j].wait()
            pltpu.async_copy(rows.at[pl.ds(j * K, K)],
                             acc.at[dstb.at[j]], ss, add=True)
        return carry

    lax.fori_loop(0, NIT2, it, 0)
    pltpu.make_async_copy(dummy, rows, ss).wait()
    plsc.subcore_barrier()
    pltpu.sync_copy(acc.at[pl.ds(sid * ZR, ZR)],
                    out_hbm.at[cid, pl.ds(sid * ZR, ZR)])


@functools.partial(
    pl.kernel,
    out_type=jax.ShapeDtypeStruct((NG, NP, 16), _f32),
    compiler_params=_sc_params,
    mesh=_mesh,
    scratch_types=[
        pltpu.VMEM((C5, K), jnp.int32),
        pltpu.VMEM((C5, K), jnp.int32),
        pltpu.VMEM((C5 * K, 16), _f32),
        pltpu.VMEM_SHARED((NP, 16), _f32),
        pltpu.VMEM_SHARED((NP, 16), _f32),
        pltpu.SemaphoreType.DMA,
        pltpu.SemaphoreType.DMA,
    ],
)
def _prop2_kernel(src2_hbm, dst2_hbm, h_hbm, zeros_hbm, out_hbm,
                  srcb, dstb, rows, tbl, acc, sg, ss):
    cid = lax.axis_index("c")
    sid = lax.axis_index("s")
    base = sid * RT1
    dummy = out_hbm.at[0, pl.ds(0, C5 * K)]

    def one_group(gi, carry):
        g = cid * (NG // NC) + gi
        pltpu.sync_copy(h_hbm.at[g, pl.ds(sid * ZR, ZR)],
                        tbl.at[pl.ds(sid * ZR, ZR)])
        pltpu.sync_copy(zeros_hbm, acc.at[pl.ds(sid * ZR, ZR)])
        plsc.subcore_barrier()

        def it(t, carry2):
            @pl.when(t > 0)
            def _():
                pltpu.make_async_copy(dummy, rows, ss).wait()

            pltpu.sync_copy(src2_hbm.at[pl.ds(base + t * C5, C5)], srcb)
            pltpu.sync_copy(dst2_hbm.at[pl.ds(base + t * C5, C5)], dstb)
            gs = [pltpu.async_copy(tbl.at[srcb.at[j]],
                                   rows.at[pl.ds(j * K, K)], sg)
                  for j in range(C5)]
            for j in range(C5):
                gs[j].wait()
                pltpu.async_copy(rows.at[pl.ds(j * K, K)],
                                 acc.at[dstb.at[j]], ss, add=True)
            return carry2

        lax.fori_loop(0, NIT1, it, 0)
        pltpu.make_async_copy(dummy, rows, ss).wait()
        plsc.subcore_barrier()
        pltpu.sync_copy(acc.at[pl.ds(sid * ZR, ZR)],
                        out_hbm.at[g, pl.ds(sid * ZR, ZR)])
        plsc.subcore_barrier()
        return carry

    lax.fori_loop(0, NG // NC, one_group, 0)


def _prep_body(degp_ref, x16_ref, xp_ref, dinv_ref):
    deg = degp_ref[0, :, 0:1] + degp_ref[1, :, 0:1] + 1.0
    dinv = lax.rsqrt(deg)
    dinv_ref[...] = dinv
    xp_ref[...] = x16_ref[...] * dinv


def _prep_call(degp, x16):
    return pl.pallas_call(
        _prep_body,
        grid=(NBLK,),
        in_specs=[
            pl.BlockSpec((NC, BN, 8), lambda i: (0, i, 0)),
            pl.BlockSpec((BN, 16), lambda i: (i, 0)),
        ],
        out_specs=[
            pl.BlockSpec((BN, 16), lambda i: (i, 0)),
            pl.BlockSpec((BN, 1), lambda i: (i, 0)),
        ],
        out_shape=[
            jax.ShapeDtypeStruct((NP, 16), _f32),
            jax.ShapeDtypeStruct((NP, 1), _f32),
        ],
    )(degp, x16)


def _h1_body(p1p_ref, xp_ref, dinv_ref, w1_ref, b1_ref, ho, o0, o1, o2, o3):
    dinv = dinv_ref[...]
    p = dinv * (p1p_ref[0] + p1p_ref[1] + xp_ref[...])
    h = jnp.dot(p, w1_ref[...], preferred_element_type=_f32) + b1_ref[...]
    hp = dinv * jnp.maximum(h, 0.0)
    for g in range(NG):
        ho[g, :, :] = hp[:, 16 * g:16 * g + 16]
    o0[...] = hp[:, 0:32]
    o1[...] = hp[:, 32:64]
    o2[...] = hp[:, 64:96]
    o3[...] = hp[:, 96:128]


def _h1_call(p1p, xp, dinv, w1p, b1r):
    return pl.pallas_call(
        _h1_body,
        grid=(NBLK,),
        in_specs=[
            pl.BlockSpec((NC, BN, 16), lambda i: (0, i, 0)),
            pl.BlockSpec((BN, 16), lambda i: (i, 0)),
            pl.BlockSpec((BN, 1), lambda i: (i, 0)),
            pl.BlockSpec((16, F_HID), lambda i: (0, 0)),
            pl.BlockSpec((1, F_HID), lambda i: (0, 0)),
        ],
        out_specs=[pl.BlockSpec((NG, BN, 16), lambda i: (0, i, 0))]
        + [pl.BlockSpec((BN, 32), lambda i: (i, 0))] * 4,
        out_shape=[jax.ShapeDtypeStruct((NG, NP, 16), _f32)]
        + [jax.ShapeDtypeStruct((NP, 32), _f32)] * 4,
    )(p1p, xp, dinv, w1p, b1r)


def _fin_body(p2_ref, h0, h1, h2, h3, dinv_ref,
              w2_ref, b2_ref, wl_ref, bl_ref, bf_ref, out_ref, acc):
    i = pl.program_id(0)

    @pl.when(i == 0)
    def _():
        acc[...] = jnp.zeros((G, F_HID), _f32)

    dinv = dinv_ref[...]
    p2 = jnp.concatenate([p2_ref[g] for g in range(NG)], axis=1)
    hp = jnp.concatenate([h0[...], h1[...], h2[...], h3[...]], axis=1)
    pf = dinv * (p2 + hp)
    h2v = jnp.dot(pf, w2_ref[...], preferred_element_type=_f32) + b2_ref[...]
    h2v = jnp.maximum(h2v, 0.0)
    h3v = lax.dot_general(h2v, wl_ref[...], (((1,), (1,)), ((), ())),
                          preferred_element_type=_f32) + bl_ref[...]
    bf = bf_ref[...]
    gids = lax.broadcasted_iota(jnp.int32, (BN, G), 1).astype(_f32)
    oh = (bf == gids).astype(_f32)
    ext = jnp.concatenate(
        [h3v, jnp.ones((BN, 1), _f32), jnp.zeros((BN, F_HID - F_OUT - 1), _f32)],
        axis=1)
    acc[...] += lax.dot_general(oh, ext, (((0,), (0,)), ((), ())),
                                preferred_element_type=_f32)

    @pl.when(i == NBLK - 1)
    def _():
        out_ref[...] = acc[:, 0:F_OUT] / jnp.maximum(acc[:, F_OUT:F_OUT + 1], 1.0)


def _fin_call(p2, hps, dinv, w2, b2r, wl, blr, bf):
    nblock = pl.BlockSpec((BN, 32), lambda i: (i, 0))
    return pl.pallas_call(
        _fin_body,
        grid=(NBLK,),
        in_specs=[pl.BlockSpec((NG, BN, 16), lambda i: (0, i, 0))]
        + [nblock] * 4 + [
            pl.BlockSpec((BN, 1), lambda i: (i, 0)),
            pl.BlockSpec((F_HID, F_HID), lambda i: (0, 0)),
            pl.BlockSpec((1, F_HID), lambda i: (0, 0)),
            pl.BlockSpec((F_OUT, F_HID), lambda i: (0, 0)),
            pl.BlockSpec((1, F_OUT), lambda i: (0, 0)),
            pl.BlockSpec((BN, 1), lambda i: (i, 0)),
        ],
        out_specs=pl.BlockSpec((G, F_OUT), lambda i: (0, 0)),
        out_shape=jax.ShapeDtypeStruct((G, F_OUT), _f32),
        scratch_shapes=[pltpu.VMEM((G, F_HID), _f32)],
    )(p2, *hps, dinv, w2, b2r, wl, blr, bf)


def kernel(x, edge_index, batch, W1, b1, W2, b2, Wlin, blin):
    src = edge_index[0]
    dst = edge_index[1]
    pad_e = EP - E
    src2 = jnp.concatenate([src, jnp.zeros((pad_e,), jnp.int32)]).reshape(EPR, K)
    dst2 = jnp.concatenate([dst, jnp.full((pad_e,), N, jnp.int32)]).reshape(EPR, K)

    x16 = jnp.zeros((NP, 16), _f32).at[:N, :7].set(x)
    w1p = jnp.zeros((16, F_HID), _f32).at[:7, :].set(W1)
    b1r = b1.reshape(1, F_HID)
    b2r = b2.reshape(1, F_HID)
    blr = blin.reshape(1, F_OUT)
    bf = jnp.full((NP, 1), float(G), _f32).at[:N, 0].set(batch.astype(_f32))

    ones8 = jnp.ones((K, 8), _f32)
    zeros8 = jnp.zeros((ZR, 8), _f32)
    zeros16 = jnp.zeros((ZR, 16), _f32)

    degp = _deg_kernel(dst2, ones8, zeros8)
    xp, dinv = _prep_call(degp, x16)
    p1p = _prop1_kernel(src2, dst2, xp, zeros16)
    ho, *hps = _h1_call(p1p, xp, dinv, w1p, b1r)
    p2 = _prop2_kernel(src2, dst2, ho, zeros16)
    return _fin_call(p2, hps, dinv, W2, b2r, Wlin, blr, bf)


# speedup vs baseline: 21.8009x; 1.3937x over previous
import functools

import jax
import jax.numpy as jnp
from jax import lax
from jax.experimental import pallas as pl
from jax.experimental.pallas import tpu as pltpu, tpu_sc as plsc

N = 50000
E = 800000
G = 64
F_HID = 128
F_OUT = 64

NP = 51200
K = 128
EP = 819200
EPR = EP // K
NC = 2
NS = 16
ZR = NP // NS

RT2 = EPR // (NC * NS)
C3 = 8
NIT2 = RT2 // C3
RT1 = EPR // NS
C5 = 10
NIT1 = RT1 // C5
NG = 8

BN = 512
NBLK = NP // BN

_mesh = plsc.VectorSubcoreMesh(core_axis_name="c", subcore_axis_name="s")
_f32 = jnp.float32
_sc_params = pltpu.CompilerParams(use_tc_tiling_on_sc=False)


@functools.partial(
    pl.kernel,
    out_type=jax.ShapeDtypeStruct((NP, 128), _f32),
    compiler_params=_sc_params,
    mesh=_mesh,
    scratch_types=[
        pltpu.VMEM((C3, K), jnp.int32),
        pltpu.VMEM((K, 8), _f32),
        pltpu.VMEM_SHARED((NP, 8), _f32),
        pltpu.SemaphoreType.DMA,
    ],
)
def _deg_kernel(dst2_hbm, ones_hbm, zeros_hbm, out_hbm, dstb, onesv, acc, ss):
    cid = lax.axis_index("c")
    sid = lax.axis_index("s")
    base = (cid * NS + sid) * RT2
    pltpu.sync_copy(zeros_hbm, acc.at[pl.ds(sid * ZR, ZR)])
    pltpu.sync_copy(ones_hbm, onesv)
    plsc.subcore_barrier()

    def it(t, carry):
        @pl.when(t > 0)
        def _():
            for _j in range(C3):
                pltpu.make_async_copy(ones_hbm, onesv, ss).wait()

        pltpu.sync_copy(dst2_hbm.at[pl.ds(base + t * C3, C3)], dstb)
        for j in range(C3):
            pltpu.async_copy(onesv, acc.at[dstb.at[j]], ss, add=True)
        return carry

    lax.fori_loop(0, NIT2, it, 0)
    for _j in range(C3):
        pltpu.make_async_copy(ones_hbm, onesv, ss).wait()
    plsc.subcore_barrier()

    @pl.when(cid == 0)
    def _():
        pltpu.sync_copy(acc.at[pl.ds(sid * ZR, ZR)],
                        out_hbm.at[pl.ds(sid * ZR, ZR), pl.ds(0, 8)])

    @pl.when(cid == 1)
    def _():
        pltpu.sync_copy(acc.at[pl.ds(sid * ZR, ZR)],
                        out_hbm.at[pl.ds(sid * ZR, ZR), pl.ds(8, 8)])


@functools.partial(
    pl.kernel,
    out_type=jax.ShapeDtypeStruct((NP, 128), _f32),
    compiler_params=_sc_params,
    mesh=_mesh,
    scratch_types=[
        pltpu.VMEM((C3, K), jnp.int32),
        pltpu.VMEM((C3, K), jnp.int32),
        pltpu.VMEM((C3 * K, 16), _f32),
        pltpu.VMEM_SHARED((NP, 16), _f32),
        pltpu.VMEM_SHARED((NP, 16), _f32),
        pltpu.SemaphoreType.DMA,
        pltpu.SemaphoreType.DMA,
    ],
)
def _prop1_kernel(src2_hbm, dst2_hbm, p_hbm, zeros_hbm, out_hbm,
                  srcb, dstb, rows, tbl, acc, sg, ss):
    cid = lax.axis_index("c")
    sid = lax.axis_index("s")
    base = (cid * NS + sid) * RT2
    dummy = out_hbm.at[pl.ds(0, C3 * K), pl.ds(0, 16)]
    pltpu.sync_copy(p_hbm.at[pl.ds(sid * ZR, ZR), pl.ds(0, 16)],
                    tbl.at[pl.ds(sid * ZR, ZR)])
    pltpu.sync_copy(zeros_hbm, acc.at[pl.ds(sid * ZR, ZR)])
    plsc.subcore_barrier()

    def it(t, carry):
        @pl.when(t > 0)
        def _():
            pltpu.make_async_copy(dummy, rows, ss).wait()

        pltpu.sync_copy(src2_hbm.at[pl.ds(base + t * C3, C3)], srcb)
        pltpu.sync_copy(dst2_hbm.at[pl.ds(base + t * C3, C3)], dstb)
        gs = [pltpu.async_copy(tbl.at[srcb.at[j]],
                               rows.at[pl.ds(j * K, K)], sg)
              for j in range(C3)]
        for j in range(C3):
            gs[j].wait()
            pltpu.async_copy(rows.at[pl.ds(j * K, K)],
                             acc.at[dstb.at[j]], ss, add=True)
        return carry

    lax.fori_loop(0, NIT2, it, 0)
    pltpu.make_async_copy(dummy, rows, ss).wait()
    plsc.subcore_barrier()

    @pl.when(cid == 0)
    def _():
        pltpu.sync_copy(acc.at[pl.ds(sid * ZR, ZR)],
                        out_hbm.at[pl.ds(sid * ZR, ZR), pl.ds(0, 16)])

    @pl.when(cid == 1)
    def _():
        pltpu.sync_copy(acc.at[pl.ds(sid * ZR, ZR)],
                        out_hbm.at[pl.ds(sid * ZR, ZR), pl.ds(16, 16)])


@functools.partial(
    pl.kernel,
    out_type=jax.ShapeDtypeStruct((NP, 128), _f32),
    compiler_params=_sc_params,
    mesh=_mesh,
    scratch_types=[
        pltpu.VMEM((C5, K), jnp.int32),
        pltpu.VMEM((C5, K), jnp.int32),
        pltpu.VMEM((C5 * K, 16), _f32),
        pltpu.VMEM_SHARED((NP, 16), _f32),
        pltpu.VMEM_SHARED((NP, 16), _f32),
        pltpu.SemaphoreType.DMA,
        pltpu.SemaphoreType.DMA,
    ],
)
def _prop2_kernel(src2_hbm, dst2_hbm, h_hbm, zeros_hbm, out_hbm,
                  srcb, dstb, rows, tbl, acc, sg, ss):
    cid = lax.axis_index("c")
    sid = lax.axis_index("s")
    base = sid * RT1
    dummy = out_hbm.at[pl.ds(0, C5 * K), pl.ds(0, 16)]

    def one_group(g):
        pltpu.sync_copy(h_hbm.at[pl.ds(sid * ZR, ZR), pl.ds(16 * g, 16)],
                        tbl.at[pl.ds(sid * ZR, ZR)])
        pltpu.sync_copy(zeros_hbm, acc.at[pl.ds(sid * ZR, ZR)])
        plsc.subcore_barrier()

        def it(t, carry2):
            @pl.when(t > 0)
            def _():
                pltpu.make_async_copy(dummy, rows, ss).wait()

            pltpu.sync_copy(src2_hbm.at[pl.ds(base + t * C5, C5)], srcb)
            pltpu.sync_copy(dst2_hbm.at[pl.ds(base + t * C5, C5)], dstb)
            gs = [pltpu.async_copy(tbl.at[srcb.at[j]],
                                   rows.at[pl.ds(j * K, K)], sg)
                  for j in range(C5)]
            for j in range(C5):
                gs[j].wait()
                pltpu.async_copy(rows.at[pl.ds(j * K, K)],
                                 acc.at[dstb.at[j]], ss, add=True)
            return carry2

        lax.fori_loop(0, NIT1, it, 0)
        pltpu.make_async_copy(dummy, rows, ss).wait()
        plsc.subcore_barrier()
        pltpu.sync_copy(acc.at[pl.ds(sid * ZR, ZR)],
                        out_hbm.at[pl.ds(sid * ZR, ZR), pl.ds(16 * g, 16)])
        plsc.subcore_barrier()

    @pl.when(cid == 0)
    def _():
        for g in range(NG // NC):
            one_group(g)

    @pl.when(cid == 1)
    def _():
        for g in range(NG // NC, NG):
            one_group(g)


def _prep_body(degd_ref, x16_ref, p_ref):
    d = degd_ref[...]
    deg = d[:, 0:1] + d[:, 8:9] + 1.0
    dinv = lax.rsqrt(deg)
    xp = x16_ref[...] * dinv
    p_ref[...] = jnp.concatenate(
        [xp, dinv, jnp.zeros((BN, 128 - 17), _f32)], axis=1)


def _prep_call(degd, x16):
    return pl.pallas_call(
        _prep_body,
        grid=(NBLK,),
        in_specs=[
            pl.BlockSpec((BN, 128), lambda i: (i, 0)),
            pl.BlockSpec((BN, 16), lambda i: (i, 0)),
        ],
        out_specs=pl.BlockSpec((BN, 128), lambda i: (i, 0)),
        out_shape=jax.ShapeDtypeStruct((NP, 128), _f32),
    )(degd, x16)


def _h1_body(o1_ref, p_ref, w1_ref, b1_ref, h_ref):
    pblk = p_ref[...]
    o1 = o1_ref[...]
    dinv = pblk[:, 16:17]
    p = dinv * (o1[:, 0:16] + o1[:, 16:32] + pblk[:, 0:16])
    h = jnp.dot(p, w1_ref[...], preferred_element_type=_f32) + b1_ref[...]
    h_ref[...] = dinv * jnp.maximum(h, 0.0)


def _h1_call(o1, p, w1p, b1r):
    return pl.pallas_call(
        _h1_body,
        grid=(NBLK,),
        in_specs=[
            pl.BlockSpec((BN, 128), lambda i: (i, 0)),
            pl.BlockSpec((BN, 128), lambda i: (i, 0)),
            pl.BlockSpec((16, F_HID), lambda i: (0, 0)),
            pl.BlockSpec((1, F_HID), lambda i: (0, 0)),
        ],
        out_specs=pl.BlockSpec((BN, 128), lambda i: (i, 0)),
        out_shape=jax.ShapeDtypeStruct((NP, 128), _f32),
    )(o1, p, w1p, b1r)


def _fin_body(p2_ref, h_ref, p_ref, w2_ref, b2_ref, wl_ref, bl_ref, bf_ref,
              out_ref, acc):
    i = pl.program_id(0)

    @pl.when(i == 0)
    def _():
        acc[...] = jnp.zeros((G, F_HID), _f32)

    dinv = p_ref[...][:, 16:17]
    pf = dinv * (p2_ref[...] + h_ref[...])
    h2v = jnp.dot(pf, w2_ref[...], preferred_element_type=_f32) + b2_ref[...]
    h2v = jnp.maximum(h2v, 0.0)
    h3v = lax.dot_general(h2v, wl_ref[...], (((1,), (1,)), ((), ())),
                          preferred_element_type=_f32) + bl_ref[...]
    bf = bf_ref[...]
    gids = lax.broadcasted_iota(jnp.int32, (BN, G), 1).astype(_f32)
    oh = (bf == gids).astype(_f32)
    ext = jnp.concatenate(
        [h3v, jnp.ones((BN, 1), _f32), jnp.zeros((BN, F_HID - F_OUT - 1), _f32)],
        axis=1)
    acc[...] += lax.dot_general(oh, ext, (((0,), (0,)), ((), ())),
                                preferred_element_type=_f32)

    @pl.when(i == NBLK - 1)
    def _():
        out_ref[...] = acc[:, 0:F_OUT] / jnp.maximum(acc[:, F_OUT:F_OUT + 1], 1.0)


def _fin_call(p2, h, p, w2, b2r, wl, blr, bf):
    blk = pl.BlockSpec((BN, 128), lambda i: (i, 0))
    return pl.pallas_call(
        _fin_body,
        grid=(NBLK,),
        in_specs=[blk, blk, blk,
            pl.BlockSpec((F_HID, F_HID), lambda i: (0, 0)),
            pl.BlockSpec((1, F_HID), lambda i: (0, 0)),
            pl.BlockSpec((F_OUT, F_HID), lambda i: (0, 0)),
            pl.BlockSpec((1, F_OUT), lambda i: (0, 0)),
            pl.BlockSpec((BN, 1), lambda i: (i, 0)),
        ],
        out_specs=pl.BlockSpec((G, F_OUT), lambda i: (0, 0)),
        out_shape=jax.ShapeDtypeStruct((G, F_OUT), _f32),
        scratch_shapes=[pltpu.VMEM((G, F_HID), _f32)],
    )(p2, h, p, w2, b2r, wl, blr, bf)


def kernel(x, edge_index, batch, W1, b1, W2, b2, Wlin, blin):
    src = edge_index[0]
    dst = edge_index[1]
    pad_e = EP - E
    src2 = jnp.concatenate([src, jnp.zeros((pad_e,), jnp.int32)]).reshape(EPR, K)
    dst2 = jnp.concatenate([dst, jnp.full((pad_e,), N, jnp.int32)]).reshape(EPR, K)

    x16 = jnp.zeros((NP, 16), _f32).at[:N, :7].set(x)
    w1p = jnp.zeros((16, F_HID), _f32).at[:7, :].set(W1)
    b1r = b1.reshape(1, F_HID)
    b2r = b2.reshape(1, F_HID)
    blr = blin.reshape(1, F_OUT)
    bf = jnp.full((NP, 1), float(G), _f32).at[:N, 0].set(batch.astype(_f32))

    ones8 = jnp.ones((K, 8), _f32)
    zeros16 = jnp.zeros((ZR, 16), _f32)
    zeros8 = jnp.zeros((ZR, 8), _f32)

    degd = _deg_kernel(dst2, ones8, zeros8)
    p = _prep_call(degd, x16)
    o1 = _prop1_kernel(src2, dst2, p, zeros16)
    h = _h1_call(o1, p, w1p, b1r)
    p2 = _prop2_kernel(src2, dst2, h, zeros16)
    return _fin_call(p2, h, p, W2, b2r, Wlin, blr, bf)


# speedup vs baseline: 23.8188x; 1.0926x over previous
import functools

import jax
import jax.numpy as jnp
from jax import lax
from jax.experimental import pallas as pl
from jax.experimental.pallas import tpu as pltpu, tpu_sc as plsc

N = 50000
E = 800000
G = 64
F_HID = 128
F_OUT = 64

NP = 51200
K = 128
EP = 819200
EPR = EP // K
NC = 2
NS = 16
ZR = NP // NS

RT2 = EPR // (NC * NS)
RT1 = EPR // NS
CK = 10
NPAIR2 = RT2 // CK // 2
NPAIR1 = RT1 // CK // 2
NIT2 = RT2 // CK
NG = 8
SDR = 2 * EPR + 2 * CK

BN = 512
NBLK = NP // BN

_mesh = plsc.VectorSubcoreMesh(core_axis_name="c", subcore_axis_name="s")
_f32 = jnp.float32
_sc_params = pltpu.CompilerParams(use_tc_tiling_on_sc=False)


@functools.partial(
    pl.kernel,
    out_type=jax.ShapeDtypeStruct((NP, 128), _f32),
    compiler_params=_sc_params,
    mesh=_mesh,
    scratch_types=[
        pltpu.VMEM((2 * CK, K), jnp.int32),
        pltpu.VMEM((K, 8), _f32),
        pltpu.VMEM_SHARED((NP, 8), _f32),
        pltpu.SemaphoreType.DMA,
    ],
)
def _deg_kernel(sd_hbm, ones_hbm, zeros_hbm, out_hbm, idxb, onesv, acc, ss):
    cid = lax.axis_index("c")
    sid = lax.axis_index("s")
    base = (cid * NS + sid) * RT2
    pltpu.sync_copy(zeros_hbm, acc.at[pl.ds(sid * ZR, ZR)])
    pltpu.sync_copy(ones_hbm, onesv)
    plsc.subcore_barrier()

    def it(t, carry):
        @pl.when(t > 0)
        def _():
            for _j in range(CK):
                pltpu.make_async_copy(ones_hbm, onesv, ss).wait()

        pltpu.sync_copy(sd_hbm.at[pl.ds(2 * (base + t * CK), 2 * CK)], idxb)
        for j in range(CK):
            pltpu.async_copy(onesv, acc.at[idxb.at[2 * j + 1]], ss, add=True)
        return carry

    lax.fori_loop(0, NIT2, it, 0)
    for _j in range(CK):
        pltpu.make_async_copy(ones_hbm, onesv, ss).wait()
    plsc.subcore_barrier()

    @pl.when(cid == 0)
    def _():
        pltpu.sync_copy(acc.at[pl.ds(sid * ZR, ZR)],
                        out_hbm.at[pl.ds(sid * ZR, ZR), pl.ds(0, 8)])

    @pl.when(cid == 1)
    def _():
        pltpu.sync_copy(acc.at[pl.ds(sid * ZR, ZR)],
                        out_hbm.at[pl.ds(sid * ZR, ZR), pl.ds(8, 8)])


def _chunk_gs(tbl, acc, rows, bank, sg, ss):
    gs = [pltpu.async_copy(tbl.at[bank.at[2 * j]],
                           rows.at[pl.ds(j * K, K)], sg)
          for j in range(CK)]
    for j in range(CK):
        gs[j].wait()
        pltpu.async_copy(rows.at[pl.ds(j * K, K)],
                         acc.at[bank.at[2 * j + 1]], ss, add=True)


def _prop_loop(sd_hbm, tbl, acc, rows, idxb, sg, ss, si, base, npair, dummy):
    chunk_bytes = sd_hbm.at[pl.ds(0, 2 * CK)]

    def prefetch(c, bank):
        pltpu.async_copy(sd_hbm.at[pl.ds(2 * base + c * 2 * CK, 2 * CK)],
                         idxb.at[bank], si)

    prefetch(0, 0)

    def it(t, carry):
        pltpu.make_async_copy(chunk_bytes, idxb.at[0], si).wait()

        @pl.when(t > 0)
        def _():
            pltpu.make_async_copy(dummy, rows, ss).wait()

        prefetch(2 * t + 1, 1)
        _chunk_gs(tbl, acc, rows, idxb.at[0], sg, ss)
        pltpu.make_async_copy(chunk_bytes, idxb.at[1], si).wait()
        pltpu.make_async_copy(dummy, rows, ss).wait()
        prefetch(2 * t + 2, 0)
        _chunk_gs(tbl, acc, rows, idxb.at[1], sg, ss)
        return carry

    lax.fori_loop(0, npair, it, 0)
    pltpu.make_async_copy(chunk_bytes, idxb.at[0], si).wait()
    pltpu.make_async_copy(dummy, rows, ss).wait()


@functools.partial(
    pl.kernel,
    out_type=jax.ShapeDtypeStruct((NP, 128), _f32),
    compiler_params=_sc_params,
    mesh=_mesh,
    scratch_types=[
        pltpu.VMEM((2, 2 * CK, K), jnp.int32),
        pltpu.VMEM((CK * K, 16), _f32),
        pltpu.VMEM_SHARED((NP, 16), _f32),
        pltpu.VMEM_SHARED((NP, 16), _f32),
        pltpu.SemaphoreType.DMA,
        pltpu.SemaphoreType.DMA,
        pltpu.SemaphoreType.DMA,
    ],
)
def _prop1_kernel(sd_hbm, p_hbm, zeros_hbm, out_hbm,
                  idxb, rows, tbl, acc, sg, ss, si):
    cid = lax.axis_index("c")
    sid = lax.axis_index("s")
    base = (cid * NS + sid) * RT2
    dummy = out_hbm.at[pl.ds(0, CK * K), pl.ds(0, 16)]
    pltpu.sync_copy(p_hbm.at[pl.ds(sid * ZR, ZR), pl.ds(0, 16)],
                    tbl.at[pl.ds(sid * ZR, ZR)])
    pltpu.sync_copy(zeros_hbm, acc.at[pl.ds(sid * ZR, ZR)])
    plsc.subcore_barrier()

    _prop_loop(sd_hbm, tbl, acc, rows, idxb, sg, ss, si, base, NPAIR2, dummy)
    plsc.subcore_barrier()

    @pl.when(cid == 0)
    def _():
        pltpu.sync_copy(acc.at[pl.ds(sid * ZR, ZR)],
                        out_hbm.at[pl.ds(sid * ZR, ZR), pl.ds(0, 16)])

    @pl.when(cid == 1)
    def _():
        pltpu.sync_copy(acc.at[pl.ds(sid * ZR, ZR)],
                        out_hbm.at[pl.ds(sid * ZR, ZR), pl.ds(16, 16)])


@functools.partial(
    pl.kernel,
    out_type=jax.ShapeDtypeStruct((NP, 128), _f32),
    compiler_params=_sc_params,
    mesh=_mesh,
    scratch_types=[
        pltpu.VMEM((2, 2 * CK, K), jnp.int32),
        pltpu.VMEM((CK * K, 16), _f32),
        pltpu.VMEM_SHARED((NP, 16), _f32),
        pltpu.VMEM_SHARED((NP, 16), _f32),
        pltpu.SemaphoreType.DMA,
        pltpu.SemaphoreType.DMA,
        pltpu.SemaphoreType.DMA,
    ],
)
def _prop2_kernel(sd_hbm, h_hbm, zeros_hbm, out_hbm,
                  idxb, rows, tbl, acc, sg, ss, si):
    cid = lax.axis_index("c")
    sid = lax.axis_index("s")
    base = sid * RT1
    dummy = out_hbm.at[pl.ds(0, CK * K), pl.ds(0, 16)]

    def one_group(g):
        pltpu.sync_copy(h_hbm.at[pl.ds(sid * ZR, ZR), pl.ds(16 * g, 16)],
                        tbl.at[pl.ds(sid * ZR, ZR)])
        pltpu.sync_copy(zeros_hbm, acc.at[pl.ds(sid * ZR, ZR)])
        plsc.subcore_barrier()
        _prop_loop(sd_hbm, tbl, acc, rows, idxb, sg, ss, si, base, NPAIR1,
                   dummy)
        plsc.subcore_barrier()
        pltpu.sync_copy(acc.at[pl.ds(sid * ZR, ZR)],
                        out_hbm.at[pl.ds(sid * ZR, ZR), pl.ds(16 * g, 16)])
        plsc.subcore_barrier()

    @pl.when(cid == 0)
    def _():
        for g in range(NG // NC):
            one_group(g)

    @pl.when(cid == 1)
    def _():
        for g in range(NG // NC, NG):
            one_group(g)


def _prep_body(degd_ref, x16_ref, p_ref):
    d = degd_ref[...]
    deg = d[:, 0:1] + d[:, 8:9] + 1.0
    dinv = lax.rsqrt(deg)
    xp = x16_ref[...] * dinv
    p_ref[...] = jnp.concatenate(
        [xp, dinv, jnp.zeros((BN, 128 - 17), _f32)], axis=1)


def _prep_call(degd, x16):
    return pl.pallas_call(
        _prep_body,
        grid=(NBLK,),
        in_specs=[
            pl.BlockSpec((BN, 128), lambda i: (i, 0)),
            pl.BlockSpec((BN, 16), lambda i: (i, 0)),
        ],
        out_specs=pl.BlockSpec((BN, 128), lambda i: (i, 0)),
        out_shape=jax.ShapeDtypeStruct((NP, 128), _f32),
    )(degd, x16)


def _h1_body(o1_ref, p_ref, w1_ref, b1_ref, h_ref):
    pblk = p_ref[...]
    o1 = o1_ref[...]
    dinv = pblk[:, 16:17]
    p = dinv * (o1[:, 0:16] + o1[:, 16:32] + pblk[:, 0:16])
    h = jnp.dot(p, w1_ref[...], preferred_element_type=_f32) + b1_ref[...]
    h_ref[...] = dinv * jnp.maximum(h, 0.0)


def _h1_call(o1, p, w1p, b1r):
    return pl.pallas_call(
        _h1_body,
        grid=(NBLK,),
        in_specs=[
            pl.BlockSpec((BN, 128), lambda i: (i, 0)),
            pl.BlockSpec((BN, 128), lambda i: (i, 0)),
            pl.BlockSpec((16, F_HID), lambda i: (0, 0)),
            pl.BlockSpec((1, F_HID), lambda i: (0, 0)),
        ],
        out_specs=pl.BlockSpec((BN, 128), lambda i: (i, 0)),
        out_shape=jax.ShapeDtypeStruct((NP, 128), _f32),
    )(o1, p, w1p, b1r)


def _fin_body(p2_ref, h_ref, p_ref, w2_ref, b2_ref, wl_ref, bl_ref, bf_ref,
              out_ref, acc):
    i = pl.program_id(0)

    @pl.when(i == 0)
    def _():
        acc[...] = jnp.zeros((G, F_HID), _f32)

    dinv = p_ref[...][:, 16:17]
    pf = dinv * (p2_ref[...] + h_ref[...])
    h2v = jnp.dot(pf, w2_ref[...], preferred_element_type=_f32) + b2_ref[...]
    h2v = jnp.maximum(h2v, 0.0)
    h3v = lax.dot_general(h2v, wl_ref[...], (((1,), (1,)), ((), ())),
                          preferred_element_type=_f32) + bl_ref[...]
    bf = bf_ref[...]
    gids = lax.broadcasted_iota(jnp.int32, (BN, G), 1).astype(_f32)
    oh = (bf == gids).astype(_f32)
    ext = jnp.concatenate(
        [h3v, jnp.ones((BN, 1), _f32), jnp.zeros((BN, F_HID - F_OUT - 1), _f32)],
        axis=1)
    acc[...] += lax.dot_general(oh, ext, (((0,), (0,)), ((), ())),
                                preferred_element_type=_f32)

    @pl.when(i == NBLK - 1)
    def _():
        out_ref[...] = acc[:, 0:F_OUT] / jnp.maximum(acc[:, F_OUT:F_OUT + 1], 1.0)


def _fin_call(p2, h, p, w2, b2r, wl, blr, bf):
    blk = pl.BlockSpec((BN, 128), lambda i: (i, 0))
    return pl.pallas_call(
        _fin_body,
        grid=(NBLK,),
        in_specs=[blk, blk, blk,
            pl.BlockSpec((F_HID, F_HID), lambda i: (0, 0)),
            pl.BlockSpec((1, F_HID), lambda i: (0, 0)),
            pl.BlockSpec((F_OUT, F_HID), lambda i: (0, 0)),
            pl.BlockSpec((1, F_OUT), lambda i: (0, 0)),
            pl.BlockSpec((BN, 1), lambda i: (i, 0)),
        ],
        out_specs=pl.BlockSpec((G, F_OUT), lambda i: (0, 0)),
        out_shape=jax.ShapeDtypeStruct((G, F_OUT), _f32),
        scratch_shapes=[pltpu.VMEM((G, F_HID), _f32)],
    )(p2, h, p, w2, b2r, wl, blr, bf)


def kernel(x, edge_index, batch, W1, b1, W2, b2, Wlin, blin):
    src = edge_index[0]
    dst = edge_index[1]
    pad_e = EP - E
    src2 = jnp.concatenate([src, jnp.zeros((pad_e,), jnp.int32)]).reshape(EPR, K)
    dst2 = jnp.concatenate([dst, jnp.full((pad_e,), N, jnp.int32)]).reshape(EPR, K)
    sd = jnp.concatenate(
        [jnp.stack([src2, dst2], axis=1).reshape(2 * EPR, K),
         jnp.zeros((2 * CK, K), jnp.int32)])

    x16 = jnp.zeros((NP, 16), _f32).at[:N, :7].set(x)
    w1p = jnp.zeros((16, F_HID), _f32).at[:7, :].set(W1)
    b1r = b1.reshape(1, F_HID)
    b2r = b2.reshape(1, F_HID)
    blr = blin.reshape(1, F_OUT)
    bf = jnp.full((NP, 1), float(G), _f32).at[:N, 0].set(batch.astype(_f32))

    ones8 = jnp.ones((K, 8), _f32)
    zeros16 = jnp.zeros((ZR, 16), _f32)
    zeros8 = jnp.zeros((ZR, 8), _f32)

    degd = _deg_kernel(sd, ones8, zeros8)
    p = _prep_call(degd, x16)
    o1 = _prop1_kernel(sd, p, zeros16)
    h = _h1_call(o1, p, w1p, b1r)
    p2 = _prop2_kernel(sd, h, zeros16)
    return _fin_call(p2, h, p, W2, b2r, Wlin, blr, bf)


# speedup vs baseline: 26.8253x; 1.1262x over previous
import functools

import jax
import jax.numpy as jnp
from jax import lax
from jax.experimental import pallas as pl
from jax.experimental.pallas import tpu as pltpu, tpu_sc as plsc

N = 50000
E = 800000
G = 64
F_HID = 128
F_OUT = 64

NP = 51200
K = 128
EP = 819200
EPR = EP // K
NC = 2
NS = 16
ZR = NP // NS

RT2 = EPR // (NC * NS)
RT1 = EPR // NS
CK = 10
NPAIR2 = RT2 // CK // 2
NPAIR1 = RT1 // CK // 2
NIT2 = RT2 // CK
NG = 8
SDR = 2 * EPR + 2 * CK

BN = 2048
NBLK = NP // BN

_mesh = plsc.VectorSubcoreMesh(core_axis_name="c", subcore_axis_name="s")
_f32 = jnp.float32
_sc_params = pltpu.CompilerParams(use_tc_tiling_on_sc=False)


@functools.partial(
    pl.kernel,
    out_type=jax.ShapeDtypeStruct((NP, 128), _f32),
    compiler_params=_sc_params,
    mesh=_mesh,
    scratch_types=[
        pltpu.VMEM((2 * CK, K), jnp.int32),
        pltpu.VMEM((K, 8), _f32),
        pltpu.VMEM_SHARED((NP, 8), _f32),
        pltpu.SemaphoreType.DMA,
    ],
)
def _deg_kernel(sd_hbm, ones_hbm, zeros_hbm, out_hbm, idxb, onesv, acc, ss):
    cid = lax.axis_index("c")
    sid = lax.axis_index("s")
    base = (cid * NS + sid) * RT2
    pltpu.sync_copy(zeros_hbm, acc.at[pl.ds(sid * ZR, ZR)])
    pltpu.sync_copy(ones_hbm, onesv)
    plsc.subcore_barrier()

    def it(t, carry):
        @pl.when(t > 0)
        def _():
            for _j in range(CK):
                pltpu.make_async_copy(ones_hbm, onesv, ss).wait()

        pltpu.sync_copy(sd_hbm.at[pl.ds(2 * (base + t * CK), 2 * CK)], idxb)
        for j in range(CK):
            pltpu.async_copy(onesv, acc.at[idxb.at[2 * j + 1]], ss, add=True)
        return carry

    lax.fori_loop(0, NIT2, it, 0)
    for _j in range(CK):
        pltpu.make_async_copy(ones_hbm, onesv, ss).wait()
    plsc.subcore_barrier()

    @pl.when(cid == 0)
    def _():
        pltpu.sync_copy(acc.at[pl.ds(sid * ZR, ZR)],
                        out_hbm.at[pl.ds(sid * ZR, ZR), pl.ds(0, 8)])

    @pl.when(cid == 1)
    def _():
        pltpu.sync_copy(acc.at[pl.ds(sid * ZR, ZR)],
                        out_hbm.at[pl.ds(sid * ZR, ZR), pl.ds(8, 8)])


def _chunk_gs(tbl, acc, rows, bank, sg, ss):
    gs = [pltpu.async_copy(tbl.at[bank.at[2 * j]],
                           rows.at[pl.ds(j * K, K)], sg)
          for j in range(CK)]
    for j in range(CK):
        gs[j].wait()
        pltpu.async_copy(rows.at[pl.ds(j * K, K)],
                         acc.at[bank.at[2 * j + 1]], ss, add=True)


def _prop_loop(sd_hbm, tbl, acc, rows, idxb, sg, ss, si, base, npair, dummy):
    chunk_bytes = sd_hbm.at[pl.ds(0, 2 * CK)]

    def prefetch(c, bank):
        pltpu.async_copy(sd_hbm.at[pl.ds(2 * base + c * 2 * CK, 2 * CK)],
                         idxb.at[bank], si)

    prefetch(0, 0)

    def it(t, carry):
        pltpu.make_async_copy(chunk_bytes, idxb.at[0], si).wait()

        @pl.when(t > 0)
        def _():
            pltpu.make_async_copy(dummy, rows, ss).wait()

        prefetch(2 * t + 1, 1)
        _chunk_gs(tbl, acc, rows, idxb.at[0], sg, ss)
        pltpu.make_async_copy(chunk_bytes, idxb.at[1], si).wait()
        pltpu.make_async_copy(dummy, rows, ss).wait()
        prefetch(2 * t + 2, 0)
        _chunk_gs(tbl, acc, rows, idxb.at[1], sg, ss)
        return carry

    lax.fori_loop(0, npair, it, 0)
    pltpu.make_async_copy(chunk_bytes, idxb.at[0], si).wait()
    pltpu.make_async_copy(dummy, rows, ss).wait()


@functools.partial(
    pl.kernel,
    out_type=jax.ShapeDtypeStruct((NP, 128), _f32),
    compiler_params=_sc_params,
    mesh=_mesh,
    scratch_types=[
        pltpu.VMEM((2, 2 * CK, K), jnp.int32),
        pltpu.VMEM((CK * K, 16), _f32),
        pltpu.VMEM_SHARED((NP, 16), _f32),
        pltpu.VMEM_SHARED((NP, 16), _f32),
        pltpu.SemaphoreType.DMA,
        pltpu.SemaphoreType.DMA,
        pltpu.SemaphoreType.DMA,
    ],
)
def _prop1_kernel(sd_hbm, p_hbm, zeros_hbm, out_hbm,
                  idxb, rows, tbl, acc, sg, ss, si):
    cid = lax.axis_index("c")
    sid = lax.axis_index("s")
    base = (cid * NS + sid) * RT2
    dummy = out_hbm.at[pl.ds(0, CK * K), pl.ds(0, 16)]
    pltpu.sync_copy(p_hbm.at[pl.ds(sid * ZR, ZR), pl.ds(0, 16)],
                    tbl.at[pl.ds(sid * ZR, ZR)])
    pltpu.sync_copy(zeros_hbm, acc.at[pl.ds(sid * ZR, ZR)])
    plsc.subcore_barrier()

    _prop_loop(sd_hbm, tbl, acc, rows, idxb, sg, ss, si, base, NPAIR2, dummy)
    plsc.subcore_barrier()

    @pl.when(cid == 0)
    def _():
        pltpu.sync_copy(acc.at[pl.ds(sid * ZR, ZR)],
                        out_hbm.at[pl.ds(sid * ZR, ZR), pl.ds(0, 16)])

    @pl.when(cid == 1)
    def _():
        pltpu.sync_copy(acc.at[pl.ds(sid * ZR, ZR)],
                        out_hbm.at[pl.ds(sid * ZR, ZR), pl.ds(16, 16)])


@functools.partial(
    pl.kernel,
    out_type=jax.ShapeDtypeStruct((NP, 128), _f32),
    compiler_params=_sc_params,
    mesh=_mesh,
    scratch_types=[
        pltpu.VMEM((2, 2 * CK, K), jnp.int32),
        pltpu.VMEM((CK * K, 16), _f32),
        pltpu.VMEM_SHARED((NP, 16), _f32),
        pltpu.VMEM_SHARED((NP, 16), _f32),
        pltpu.SemaphoreType.DMA,
        pltpu.SemaphoreType.DMA,
        pltpu.SemaphoreType.DMA,
    ],
)
def _prop2_kernel(sd_hbm, h_hbm, zeros_hbm, out_hbm,
                  idxb, rows, tbl, acc, sg, ss, si):
    cid = lax.axis_index("c")
    sid = lax.axis_index("s")
    base = sid * RT1
    dummy = out_hbm.at[pl.ds(0, CK * K), pl.ds(0, 16)]

    def one_group(g):
        pltpu.sync_copy(h_hbm.at[pl.ds(sid * ZR, ZR), pl.ds(16 * g, 16)],
                        tbl.at[pl.ds(sid * ZR, ZR)])
        pltpu.sync_copy(zeros_hbm, acc.at[pl.ds(sid * ZR, ZR)])
        plsc.subcore_barrier()
        _prop_loop(sd_hbm, tbl, acc, rows, idxb, sg, ss, si, base, NPAIR1,
                   dummy)
        plsc.subcore_barrier()
        pltpu.sync_copy(acc.at[pl.ds(sid * ZR, ZR)],
                        out_hbm.at[pl.ds(sid * ZR, ZR), pl.ds(16 * g, 16)])
        plsc.subcore_barrier()

    @pl.when(cid == 0)
    def _():
        for g in range(NG // NC):
            one_group(g)

    @pl.when(cid == 1)
    def _():
        for g in range(NG // NC, NG):
            one_group(g)


def _prep_body(degd_ref, x16_ref, p_ref):
    d = degd_ref[...]
    deg = d[:, 0:1] + d[:, 8:9] + 1.0
    dinv = lax.rsqrt(deg)
    xp = x16_ref[...] * dinv
    p_ref[...] = jnp.concatenate(
        [xp, dinv, jnp.zeros((BN, 128 - 17), _f32)], axis=1)


def _prep_call(degd, x16):
    return pl.pallas_call(
        _prep_body,
        grid=(NBLK,),
        in_specs=[
            pl.BlockSpec((BN, 128), lambda i: (i, 0)),
            pl.BlockSpec((BN, 16), lambda i: (i, 0)),
        ],
        out_specs=pl.BlockSpec((BN, 128), lambda i: (i, 0)),
        out_shape=jax.ShapeDtypeStruct((NP, 128), _f32),
    )(degd, x16)


def _h1_body(o1_ref, p_ref, w1_ref, b1_ref, h_ref):
    pblk = p_ref[...]
    o1 = o1_ref[...]
    dinv = pblk[:, 16:17]
    p = dinv * (o1[:, 0:16] + o1[:, 16:32] + pblk[:, 0:16])
    h = jnp.dot(p, w1_ref[...], preferred_element_type=_f32) + b1_ref[...]
    h_ref[...] = dinv * jnp.maximum(h, 0.0)


def _h1_call(o1, p, w1p, b1r):
    return pl.pallas_call(
        _h1_body,
        grid=(NBLK,),
        in_specs=[
            pl.BlockSpec((BN, 128), lambda i: (i, 0)),
            pl.BlockSpec((BN, 128), lambda i: (i, 0)),
            pl.BlockSpec((16, F_HID), lambda i: (0, 0)),
            pl.BlockSpec((1, F_HID), lambda i: (0, 0)),
        ],
        out_specs=pl.BlockSpec((BN, 128), lambda i: (i, 0)),
        out_shape=jax.ShapeDtypeStruct((NP, 128), _f32),
    )(o1, p, w1p, b1r)


def _fin_body(p2_ref, h_ref, p_ref, w2_ref, b2_ref, wl_ref, bl_ref, bf_ref,
              out_ref, acc):
    i = pl.program_id(0)

    @pl.when(i == 0)
    def _():
        acc[...] = jnp.zeros((G, F_HID), _f32)

    dinv = p_ref[...][:, 16:17]
    pf = dinv * (p2_ref[...] + h_ref[...])
    h2v = jnp.dot(pf, w2_ref[...], preferred_element_type=_f32) + b2_ref[...]
    h2v = jnp.maximum(h2v, 0.0)
    h3v = lax.dot_general(h2v, wl_ref[...], (((1,), (1,)), ((), ())),
                          preferred_element_type=_f32) + bl_ref[...]
    bf = bf_ref[...]
    gids = lax.broadcasted_iota(jnp.int32, (BN, G), 1).astype(_f32)
    oh = (bf == gids).astype(_f32)
    ext = jnp.concatenate(
        [h3v, jnp.ones((BN, 1), _f32), jnp.zeros((BN, F_HID - F_OUT - 1), _f32)],
        axis=1)
    acc[...] += lax.dot_general(oh, ext, (((0,), (0,)), ((), ())),
                                preferred_element_type=_f32)

    @pl.when(i == NBLK - 1)
    def _():
        out_ref[...] = acc[:, 0:F_OUT] / jnp.maximum(acc[:, F_OUT:F_OUT + 1], 1.0)


def _fin_call(p2, h, p, w2, b2r, wl, blr, bf):
    blk = pl.BlockSpec((BN, 128), lambda i: (i, 0))
    return pl.pallas_call(
        _fin_body,
        grid=(NBLK,),
        in_specs=[blk, blk, blk,
            pl.BlockSpec((F_HID, F_HID), lambda i: (0, 0)),
            pl.BlockSpec((1, F_HID), lambda i: (0, 0)),
            pl.BlockSpec((F_OUT, F_HID), lambda i: (0, 0)),
            pl.BlockSpec((1, F_OUT), lambda i: (0, 0)),
            pl.BlockSpec((BN, 1), lambda i: (i, 0)),
        ],
        out_specs=pl.BlockSpec((G, F_OUT), lambda i: (0, 0)),
        out_shape=jax.ShapeDtypeStruct((G, F_OUT), _f32),
        scratch_shapes=[pltpu.VMEM((G, F_HID), _f32)],
    )(p2, h, p, w2, b2r, wl, blr, bf)


def kernel(x, edge_index, batch, W1, b1, W2, b2, Wlin, blin):
    src = edge_index[0]
    dst = edge_index[1]
    pad_e = EP - E
    src2 = jnp.concatenate([src, jnp.zeros((pad_e,), jnp.int32)]).reshape(EPR, K)
    dst2 = jnp.concatenate([dst, jnp.full((pad_e,), N, jnp.int32)]).reshape(EPR, K)
    sd = jnp.concatenate(
        [jnp.stack([src2, dst2], axis=1).reshape(2 * EPR, K),
         jnp.zeros((2 * CK, K), jnp.int32)])

    x16 = jnp.zeros((NP, 16), _f32).at[:N, :7].set(x)
    w1p = jnp.zeros((16, F_HID), _f32).at[:7, :].set(W1)
    b1r = b1.reshape(1, F_HID)
    b2r = b2.reshape(1, F_HID)
    blr = blin.reshape(1, F_OUT)
    bf = jnp.full((NP, 1), float(G), _f32).at[:N, 0].set(batch.astype(_f32))

    ones8 = jnp.ones((K, 8), _f32)
    zeros16 = jnp.zeros((ZR, 16), _f32)
    zeros8 = jnp.zeros((ZR, 8), _f32)

    degd = _deg_kernel(sd, ones8, zeros8)
    p = _prep_call(degd, x16)
    o1 = _prop1_kernel(sd, p, zeros16)
    h = _h1_call(o1, p, w1p, b1r)
    p2 = _prop2_kernel(sd, h, zeros16)
    return _fin_call(p2, h, p, W2, b2r, Wlin, blr, bf)


# speedup vs baseline: 27.3503x; 1.0196x over previous
import functools

import jax
import jax.numpy as jnp
from jax import lax
from jax.experimental import pallas as pl
from jax.experimental.pallas import tpu as pltpu, tpu_sc as plsc

N = 50000
E = 800000
G = 64
F_HID = 128
F_OUT = 64

NP = 51200
K = 128
EP = 819200
EPR = EP // K
NC = 2
NS = 16
ZR = NP // NS

RT2 = EPR // (NC * NS)
RT1 = EPR // NS
CK = 10
NPAIR2 = RT2 // CK // 2
NPAIR1 = RT1 // CK // 2
NIT2 = RT2 // CK
NG = 8
SDR = 2 * EPR + 2 * CK

BN = 3200
NBLK = NP // BN

_mesh = plsc.VectorSubcoreMesh(core_axis_name="c", subcore_axis_name="s")
_f32 = jnp.float32
_sc_params = pltpu.CompilerParams(use_tc_tiling_on_sc=False)


@functools.partial(
    pl.kernel,
    out_type=jax.ShapeDtypeStruct((NP, 128), _f32),
    compiler_params=_sc_params,
    mesh=_mesh,
    scratch_types=[
        pltpu.VMEM((2 * CK, K), jnp.int32),
        pltpu.VMEM((K, 8), _f32),
        pltpu.VMEM_SHARED((NP, 8), _f32),
        pltpu.SemaphoreType.DMA,
    ],
)
def _deg_kernel(sd_hbm, ones_hbm, zeros_hbm, out_hbm, idxb, onesv, acc, ss):
    cid = lax.axis_index("c")
    sid = lax.axis_index("s")
    base = (cid * NS + sid) * RT2
    pltpu.sync_copy(zeros_hbm, acc.at[pl.ds(sid * ZR, ZR)])
    pltpu.sync_copy(ones_hbm, onesv)
    plsc.subcore_barrier()

    def it(t, carry):
        @pl.when(t > 0)
        def _():
            for _j in range(CK):
                pltpu.make_async_copy(ones_hbm, onesv, ss).wait()

        pltpu.sync_copy(sd_hbm.at[pl.ds(2 * (base + t * CK), 2 * CK)], idxb)
        for j in range(CK):
            pltpu.async_copy(onesv, acc.at[idxb.at[2 * j + 1]], ss, add=True)
        return carry

    lax.fori_loop(0, NIT2, it, 0)
    for _j in range(CK):
        pltpu.make_async_copy(ones_hbm, onesv, ss).wait()
    plsc.subcore_barrier()

    @pl.when(cid == 0)
    def _():
        pltpu.sync_copy(acc.at[pl.ds(sid * ZR, ZR)],
                        out_hbm.at[pl.ds(sid * ZR, ZR), pl.ds(0, 8)])

    @pl.when(cid == 1)
    def _():
        pltpu.sync_copy(acc.at[pl.ds(sid * ZR, ZR)],
                        out_hbm.at[pl.ds(sid * ZR, ZR), pl.ds(8, 8)])


def _chunk_gs(tbl, acc, rows, bank, sg, ss):
    gs = [pltpu.async_copy(tbl.at[bank.at[2 * j]],
                           rows.at[pl.ds(j * K, K)], sg)
          for j in range(CK)]
    for j in range(CK):
        gs[j].wait()
        pltpu.async_copy(rows.at[pl.ds(j * K, K)],
                         acc.at[bank.at[2 * j + 1]], ss, add=True)


def _prop_loop(sd_hbm, tbl, acc, rows, idxb, sg, ss, si, base, npair, dummy):
    chunk_bytes = sd_hbm.at[pl.ds(0, 2 * CK)]

    def prefetch(c, bank):
        pltpu.async_copy(sd_hbm.at[pl.ds(2 * base + c * 2 * CK, 2 * CK)],
                         idxb.at[bank], si)

    prefetch(0, 0)

    def it(t, carry):
        pltpu.make_async_copy(chunk_bytes, idxb.at[0], si).wait()

        @pl.when(t > 0)
        def _():
            pltpu.make_async_copy(dummy, rows, ss).wait()

        prefetch(2 * t + 1, 1)
        _chunk_gs(tbl, acc, rows, idxb.at[0], sg, ss)
        pltpu.make_async_copy(chunk_bytes, idxb.at[1], si).wait()
        pltpu.make_async_copy(dummy, rows, ss).wait()
        prefetch(2 * t + 2, 0)
        _chunk_gs(tbl, acc, rows, idxb.at[1], sg, ss)
        return carry

    lax.fori_loop(0, npair, it, 0)
    pltpu.make_async_copy(chunk_bytes, idxb.at[0], si).wait()
    pltpu.make_async_copy(dummy, rows, ss).wait()


@functools.partial(
    pl.kernel,
    out_type=jax.ShapeDtypeStruct((NP, 128), _f32),
    compiler_params=_sc_params,
    mesh=_mesh,
    scratch_types=[
        pltpu.VMEM((2, 2 * CK, K), jnp.int32),
        pltpu.VMEM((CK * K, 16), _f32),
        pltpu.VMEM_SHARED((NP, 16), _f32),
        pltpu.VMEM_SHARED((NP, 16), _f32),
        pltpu.SemaphoreType.DMA,
        pltpu.SemaphoreType.DMA,
        pltpu.SemaphoreType.DMA,
    ],
)
def _prop1_kernel(sd_hbm, p_hbm, zeros_hbm, out_hbm,
                  idxb, rows, tbl, acc, sg, ss, si):
    cid = lax.axis_index("c")
    sid = lax.axis_index("s")
    base = (cid * NS + sid) * RT2
    dummy = out_hbm.at[pl.ds(0, CK * K), pl.ds(0, 16)]
    pltpu.sync_copy(p_hbm.at[pl.ds(sid * ZR, ZR), pl.ds(0, 16)],
                    tbl.at[pl.ds(sid * ZR, ZR)])
    pltpu.sync_copy(zeros_hbm, acc.at[pl.ds(sid * ZR, ZR)])
    plsc.subcore_barrier()

    _prop_loop(sd_hbm, tbl, acc, rows, idxb, sg, ss, si, base, NPAIR2, dummy)
    plsc.subcore_barrier()

    @pl.when(cid == 0)
    def _():
        pltpu.sync_copy(acc.at[pl.ds(sid * ZR, ZR)],
                        out_hbm.at[pl.ds(sid * ZR, ZR), pl.ds(0, 16)])

    @pl.when(cid == 1)
    def _():
        pltpu.sync_copy(acc.at[pl.ds(sid * ZR, ZR)],
                        out_hbm.at[pl.ds(sid * ZR, ZR), pl.ds(16, 16)])


@functools.partial(
    pl.kernel,
    out_type=jax.ShapeDtypeStruct((NP, 128), _f32),
    compiler_params=_sc_params,
    mesh=_mesh,
    scratch_types=[
        pltpu.VMEM((2, 2 * CK, K), jnp.int32),
        pltpu.VMEM((CK * K, 16), _f32),
        pltpu.VMEM_SHARED((NP, 16), _f32),
        pltpu.VMEM_SHARED((NP, 16), _f32),
        pltpu.SemaphoreType.DMA,
        pltpu.SemaphoreType.DMA,
        pltpu.SemaphoreType.DMA,
    ],
)
def _prop2_kernel(sd_hbm, h_hbm, zeros_hbm, out_hbm,
                  idxb, rows, tbl, acc, sg, ss, si):
    cid = lax.axis_index("c")
    sid = lax.axis_index("s")
    base = sid * RT1
    dummy = out_hbm.at[pl.ds(0, CK * K), pl.ds(0, 16)]

    def one_group(g):
        pltpu.sync_copy(h_hbm.at[pl.ds(sid * ZR, ZR), pl.ds(16 * g, 16)],
                        tbl.at[pl.ds(sid * ZR, ZR)])
        pltpu.sync_copy(zeros_hbm, acc.at[pl.ds(sid * ZR, ZR)])
        plsc.subcore_barrier()
        _prop_loop(sd_hbm, tbl, acc, rows, idxb, sg, ss, si, base, NPAIR1,
                   dummy)
        plsc.subcore_barrier()
        pltpu.sync_copy(acc.at[pl.ds(sid * ZR, ZR)],
                        out_hbm.at[pl.ds(sid * ZR, ZR), pl.ds(16 * g, 16)])
        plsc.subcore_barrier()

    @pl.when(cid == 0)
    def _():
        for g in range(NG // NC):
            one_group(g)

    @pl.when(cid == 1)
    def _():
        for g in range(NG // NC, NG):
            one_group(g)


def _prep_body(degd_ref, x16_ref, p_ref):
    d = degd_ref[...]
    deg = d[:, 0:1] + d[:, 8:9] + 1.0
    dinv = lax.rsqrt(deg)
    xp = x16_ref[...] * dinv
    p_ref[...] = jnp.concatenate(
        [xp, dinv, jnp.zeros((BN, 128 - 17), _f32)], axis=1)


def _prep_call(degd, x16):
    return pl.pallas_call(
        _prep_body,
        grid=(NBLK,),
        in_specs=[
            pl.BlockSpec((BN, 128), lambda i: (i, 0)),
            pl.BlockSpec((BN, 16), lambda i: (i, 0)),
        ],
        out_specs=pl.BlockSpec((BN, 128), lambda i: (i, 0)),
        out_shape=jax.ShapeDtypeStruct((NP, 128), _f32),
    )(degd, x16)


def _h1_body(o1_ref, p_ref, w1_ref, b1_ref, h_ref):
    pblk = p_ref[...]
    o1 = o1_ref[...]
    dinv = pblk[:, 16:17]
    p = dinv * (o1[:, 0:16] + o1[:, 16:32] + pblk[:, 0:16])
    h = jnp.dot(p, w1_ref[...], preferred_element_type=_f32) + b1_ref[...]
    h_ref[...] = dinv * jnp.maximum(h, 0.0)


def _h1_call(o1, p, w1p, b1r):
    return pl.pallas_call(
        _h1_body,
        grid=(NBLK,),
        in_specs=[
            pl.BlockSpec((BN, 128), lambda i: (i, 0)),
            pl.BlockSpec((BN, 128), lambda i: (i, 0)),
            pl.BlockSpec((16, F_HID), lambda i: (0, 0)),
            pl.BlockSpec((1, F_HID), lambda i: (0, 0)),
        ],
        out_specs=pl.BlockSpec((BN, 128), lambda i: (i, 0)),
        out_shape=jax.ShapeDtypeStruct((NP, 128), _f32),
    )(o1, p, w1p, b1r)


def _fin_body(p2_ref, h_ref, p_ref, w2_ref, b2_ref, wl_ref, bl_ref, bf_ref,
              out_ref, acc):
    i = pl.program_id(0)

    @pl.when(i == 0)
    def _():
        acc[...] = jnp.zeros((G, F_HID), _f32)

    dinv = p_ref[...][:, 16:17]
    pf = dinv * (p2_ref[...] + h_ref[...])
    h2v = jnp.dot(pf, w2_ref[...], preferred_element_type=_f32) + b2_ref[...]
    h2v = jnp.maximum(h2v, 0.0)
    h3v = lax.dot_general(h2v, wl_ref[...], (((1,), (1,)), ((), ())),
                          preferred_element_type=_f32) + bl_ref[...]
    bf = bf_ref[...]
    gids = lax.broadcasted_iota(jnp.int32, (BN, G), 1).astype(_f32)
    oh = (bf == gids).astype(_f32)
    ext = jnp.concatenate(
        [h3v, jnp.ones((BN, 1), _f32), jnp.zeros((BN, F_HID - F_OUT - 1), _f32)],
        axis=1)
    acc[...] += lax.dot_general(oh, ext, (((0,), (0,)), ((), ())),
                                preferred_element_type=_f32)

    @pl.when(i == NBLK - 1)
    def _():
        out_ref[...] = acc[:, 0:F_OUT] / jnp.maximum(acc[:, F_OUT:F_OUT + 1], 1.0)


def _fin_call(p2, h, p, w2, b2r, wl, blr, bf):
    blk = pl.BlockSpec((BN, 128), lambda i: (i, 0))
    return pl.pallas_call(
        _fin_body,
        grid=(NBLK,),
        in_specs=[blk, blk, blk,
            pl.BlockSpec((F_HID, F_HID), lambda i: (0, 0)),
            pl.BlockSpec((1, F_HID), lambda i: (0, 0)),
            pl.BlockSpec((F_OUT, F_HID), lambda i: (0, 0)),
            pl.BlockSpec((1, F_OUT), lambda i: (0, 0)),
            pl.BlockSpec((BN, 1), lambda i: (i, 0)),
        ],
        out_specs=pl.BlockSpec((G, F_OUT), lambda i: (0, 0)),
        out_shape=jax.ShapeDtypeStruct((G, F_OUT), _f32),
        scratch_shapes=[pltpu.VMEM((G, F_HID), _f32)],
    )(p2, h, p, w2, b2r, wl, blr, bf)


def kernel(x, edge_index, batch, W1, b1, W2, b2, Wlin, blin):
    src = edge_index[0]
    dst = edge_index[1]
    pad_e = EP - E
    src2 = jnp.concatenate([src, jnp.zeros((pad_e,), jnp.int32)]).reshape(EPR, K)
    dst2 = jnp.concatenate([dst, jnp.full((pad_e,), N, jnp.int32)]).reshape(EPR, K)
    sd = jnp.concatenate(
        [jnp.stack([src2, dst2], axis=1).reshape(2 * EPR, K),
         jnp.zeros((2 * CK, K), jnp.int32)])

    x16 = jnp.zeros((NP, 16), _f32).at[:N, :7].set(x)
    w1p = jnp.zeros((16, F_HID), _f32).at[:7, :].set(W1)
    b1r = b1.reshape(1, F_HID)
    b2r = b2.reshape(1, F_HID)
    blr = blin.reshape(1, F_OUT)
    bf = jnp.full((NP, 1), float(G), _f32).at[:N, 0].set(batch.astype(_f32))

    ones8 = jnp.ones((K, 8), _f32)
    zeros16 = jnp.zeros((ZR, 16), _f32)
    zeros8 = jnp.zeros((ZR, 8), _f32)

    degd = _deg_kernel(sd, ones8, zeros8)
    p = _prep_call(degd, x16)
    o1 = _prop1_kernel(sd, p, zeros16)
    h = _h1_call(o1, p, w1p, b1r)
    p2 = _prop2_kernel(sd, h, zeros16)
    return _fin_call(p2, h, p, W2, b2r, Wlin, blr, bf)


# speedup vs baseline: 30.5792x; 1.1181x over previous
import functools

import jax
import jax.numpy as jnp
from jax import lax
from jax.experimental import pallas as pl
from jax.experimental.pallas import tpu as pltpu, tpu_sc as plsc

N = 50000
E = 800000
G = 64
F_HID = 128
F_OUT = 64

NP = 51200
K = 128
EP = 819200
EPR = EP // K
NC = 2
NS = 16
ZR = NP // NS

RT2 = EPR // (NC * NS)
RT1 = EPR // NS
CK = 5
NQ2 = RT2 // CK // 4
NQ1 = RT1 // CK // 4
NIT2 = RT2 // (2 * CK)
NG = 8
SDR = 2 * EPR + 4 * CK

BN = 3200
NBLK = NP // BN

_mesh = plsc.VectorSubcoreMesh(core_axis_name="c", subcore_axis_name="s")
_f32 = jnp.float32
_sc_params = pltpu.CompilerParams(use_tc_tiling_on_sc=False)


@functools.partial(
    pl.kernel,
    out_type=jax.ShapeDtypeStruct((NP, 128), _f32),
    compiler_params=_sc_params,
    mesh=_mesh,
    scratch_types=[
        pltpu.VMEM((4 * CK, K), jnp.int32),
        pltpu.VMEM((K, 8), _f32),
        pltpu.VMEM_SHARED((NP, 8), _f32),
        pltpu.SemaphoreType.DMA,
    ],
)
def _deg_kernel(sd_hbm, ones_hbm, zeros_hbm, out_hbm, idxb, onesv, acc, ss):
    cid = lax.axis_index("c")
    sid = lax.axis_index("s")
    base = (cid * NS + sid) * RT2
    pltpu.sync_copy(zeros_hbm, acc.at[pl.ds(sid * ZR, ZR)])
    pltpu.sync_copy(ones_hbm, onesv)
    plsc.subcore_barrier()

    def it(t, carry):
        @pl.when(t > 0)
        def _():
            for _j in range(2 * CK):
                pltpu.make_async_copy(ones_hbm, onesv, ss).wait()

        pltpu.sync_copy(sd_hbm.at[pl.ds(2 * base + t * 4 * CK, 4 * CK)], idxb)
        for j in range(2 * CK):
            pltpu.async_copy(onesv, acc.at[idxb.at[2 * j + 1]], ss, add=True)
        return carry

    lax.fori_loop(0, NIT2, it, 0)
    for _j in range(2 * CK):
        pltpu.make_async_copy(ones_hbm, onesv, ss).wait()
    plsc.subcore_barrier()

    @pl.when(cid == 0)
    def _():
        pltpu.sync_copy(acc.at[pl.ds(sid * ZR, ZR)],
                        out_hbm.at[pl.ds(sid * ZR, ZR), pl.ds(0, 8)])

    @pl.when(cid == 1)
    def _():
        pltpu.sync_copy(acc.at[pl.ds(sid * ZR, ZR)],
                        out_hbm.at[pl.ds(sid * ZR, ZR), pl.ds(8, 8)])


def _chunk_gs(tbl, acc, rows, bank, sg, ss):
    gs = [pltpu.async_copy(tbl.at[bank.at[2 * j]],
                           rows.at[pl.ds(j * K, K)], sg)
          for j in range(CK)]
    for j in range(CK):
        gs[j].wait()
        pltpu.async_copy(rows.at[pl.ds(j * K, K)],
                         acc.at[bank.at[2 * j + 1]], ss, add=True)


def _prop_loop(sd_hbm, tbl, acc, r0, r1, idxb, sg, ss0, ss1, si, base, nquad,
               d0, d1):
    chunk_bytes = sd_hbm.at[pl.ds(0, 2 * CK)]

    def prefetch(c, bank):
        pltpu.async_copy(sd_hbm.at[pl.ds(2 * base + c * 2 * CK, 2 * CK)],
                         idxb.at[bank], si)

    def iwait(bank):
        pltpu.make_async_copy(chunk_bytes, idxb.at[bank], si).wait()

    prefetch(0, 0)
    prefetch(1, 1)

    def it(t, carry):
        iwait(0)

        @pl.when(t > 0)
        def _():
            pltpu.make_async_copy(d0, r0, ss0).wait()

        prefetch(4 * t + 2, 2)
        _chunk_gs(tbl, acc, r0, idxb.at[0], sg, ss0)
        iwait(1)

        @pl.when(t > 0)
        def _():
            pltpu.make_async_copy(d1, r1, ss1).wait()

        prefetch(4 * t + 3, 3)
        _chunk_gs(tbl, acc, r1, idxb.at[1], sg, ss1)
        iwait(2)
        pltpu.make_async_copy(d0, r0, ss0).wait()
        prefetch(4 * t + 4, 0)
        _chunk_gs(tbl, acc, r0, idxb.at[2], sg, ss0)
        iwait(3)
        pltpu.make_async_copy(d1, r1, ss1).wait()
        prefetch(4 * t + 5, 1)
        _chunk_gs(tbl, acc, r1, idxb.at[3], sg, ss1)
        return carry

    lax.fori_loop(0, nquad, it, 0)
    iwait(0)
    iwait(1)
    pltpu.make_async_copy(d0, r0, ss0).wait()
    pltpu.make_async_copy(d1, r1, ss1).wait()


@functools.partial(
    pl.kernel,
    out_type=jax.ShapeDtypeStruct((NP, 128), _f32),
    compiler_params=_sc_params,
    mesh=_mesh,
    scratch_types=[
        pltpu.VMEM((4, 2 * CK, K), jnp.int32),
        pltpu.VMEM((CK * K, 16), _f32),
        pltpu.VMEM((CK * K, 16), _f32),
        pltpu.VMEM_SHARED((NP, 16), _f32),
        pltpu.VMEM_SHARED((NP, 16), _f32),
        pltpu.SemaphoreType.DMA,
        pltpu.SemaphoreType.DMA,
        pltpu.SemaphoreType.DMA,
        pltpu.SemaphoreType.DMA,
    ],
)
def _prop1_kernel(sd_hbm, p_hbm, zeros_hbm, out_hbm,
                  idxb, r0, r1, tbl, acc, sg, ss0, ss1, si):
    cid = lax.axis_index("c")
    sid = lax.axis_index("s")
    base = (cid * NS + sid) * RT2
    dummy = out_hbm.at[pl.ds(0, CK * K), pl.ds(0, 16)]
    pltpu.sync_copy(p_hbm.at[pl.ds(sid * ZR, ZR), pl.ds(0, 16)],
                    tbl.at[pl.ds(sid * ZR, ZR)])
    pltpu.sync_copy(zeros_hbm, acc.at[pl.ds(sid * ZR, ZR)])
    plsc.subcore_barrier()

    _prop_loop(sd_hbm, tbl, acc, r0, r1, idxb, sg, ss0, ss1, si, base, NQ2,
               dummy, dummy)
    plsc.subcore_barrier()

    @pl.when(cid == 0)
    def _():
        pltpu.sync_copy(acc.at[pl.ds(sid * ZR, ZR)],
                        out_hbm.at[pl.ds(sid * ZR, ZR), pl.ds(0, 16)])

    @pl.when(cid == 1)
    def _():
        pltpu.sync_copy(acc.at[pl.ds(sid * ZR, ZR)],
                        out_hbm.at[pl.ds(sid * ZR, ZR), pl.ds(16, 16)])


@functools.partial(
    pl.kernel,
    out_type=jax.ShapeDtypeStruct((NP, 128), _f32),
    compiler_params=_sc_params,
    mesh=_mesh,
    scratch_types=[
        pltpu.VMEM((4, 2 * CK, K), jnp.int32),
        pltpu.VMEM((CK * K, 16), _f32),
        pltpu.VMEM((CK * K, 16), _f32),
        pltpu.VMEM_SHARED((NP, 16), _f32),
        pltpu.VMEM_SHARED((NP, 16), _f32),
        pltpu.SemaphoreType.DMA,
        pltpu.SemaphoreType.DMA,
        pltpu.SemaphoreType.DMA,
        pltpu.SemaphoreType.DMA,
    ],
)
def _prop2_kernel(sd_hbm, h_hbm, zeros_hbm, out_hbm,
                  idxb, r0, r1, tbl, acc, sg, ss0, ss1, si):
    cid = lax.axis_index("c")
    sid = lax.axis_index("s")
    base = sid * RT1
    dummy = out_hbm.at[pl.ds(0, CK * K), pl.ds(0, 16)]

    def one_group(g):
        pltpu.sync_copy(h_hbm.at[pl.ds(sid * ZR, ZR), pl.ds(16 * g, 16)],
                        tbl.at[pl.ds(sid * ZR, ZR)])
        pltpu.sync_copy(zeros_hbm, acc.at[pl.ds(sid * ZR, ZR)])
        plsc.subcore_barrier()
        _prop_loop(sd_hbm, tbl, acc, r0, r1, idxb, sg, ss0, ss1, si, base,
                   NQ1, dummy, dummy)
        plsc.subcore_barrier()
        pltpu.sync_copy(acc.at[pl.ds(sid * ZR, ZR)],
                        out_hbm.at[pl.ds(sid * ZR, ZR), pl.ds(16 * g, 16)])
        plsc.subcore_barrier()

    @pl.when(cid == 0)
    def _():
        for g in range(NG // NC):
            one_group(g)

    @pl.when(cid == 1)
    def _():
        for g in range(NG // NC, NG):
            one_group(g)


def _prep_body(degd_ref, x16_ref, p_ref):
    d = degd_ref[...]
    deg = d[:, 0:1] + d[:, 8:9] + 1.0
    dinv = lax.rsqrt(deg)
    xp = x16_ref[...] * dinv
    p_ref[...] = jnp.concatenate(
        [xp, dinv, jnp.zeros((BN, 128 - 17), _f32)], axis=1)


def _prep_call(degd, x16):
    return pl.pallas_call(
        _prep_body,
        grid=(NBLK,),
        in_specs=[
            pl.BlockSpec((BN, 128), lambda i: (i, 0)),
            pl.BlockSpec((BN, 16), lambda i: (i, 0)),
        ],
        out_specs=pl.BlockSpec((BN, 128), lambda i: (i, 0)),
        out_shape=jax.ShapeDtypeStruct((NP, 128), _f32),
    )(degd, x16)


def _h1_body(o1_ref, p_ref, w1_ref, b1_ref, h_ref):
    pblk = p_ref[...]
    o1 = o1_ref[...]
    dinv = pblk[:, 16:17]
    p = dinv * (o1[:, 0:16] + o1[:, 16:32] + pblk[:, 0:16])
    h = jnp.dot(p, w1_ref[...], preferred_element_type=_f32) + b1_ref[...]
    h_ref[...] = dinv * jnp.maximum(h, 0.0)


def _h1_call(o1, p, w1p, b1r):
    return pl.pallas_call(
        _h1_body,
        grid=(NBLK,),
        in_specs=[
            pl.BlockSpec((BN, 128), lambda i: (i, 0)),
            pl.BlockSpec((BN, 128), lambda i: (i, 0)),
            pl.BlockSpec((16, F_HID), lambda i: (0, 0)),
            pl.BlockSpec((1, F_HID), lambda i: (0, 0)),
        ],
        out_specs=pl.BlockSpec((BN, 128), lambda i: (i, 0)),
        out_shape=jax.ShapeDtypeStruct((NP, 128), _f32),
    )(o1, p, w1p, b1r)


def _fin_body(p2_ref, h_ref, p_ref, w2_ref, b2_ref, wl_ref, bl_ref, bf_ref,
              out_ref, acc):
    i = pl.program_id(0)

    @pl.when(i == 0)
    def _():
        acc[...] = jnp.zeros((G, F_HID), _f32)

    dinv = p_ref[...][:, 16:17]
    pf = dinv * (p2_ref[...] + h_ref[...])
    h2v = jnp.dot(pf, w2_ref[...], preferred_element_type=_f32) + b2_ref[...]
    h2v = jnp.maximum(h2v, 0.0)
    h3v = lax.dot_general(h2v, wl_ref[...], (((1,), (1,)), ((), ())),
                          preferred_element_type=_f32) + bl_ref[...]
    bf = bf_ref[...]
    gids = lax.broadcasted_iota(jnp.int32, (BN, G), 1).astype(_f32)
    oh = (bf == gids).astype(_f32)
    ext = jnp.concatenate(
        [h3v, jnp.ones((BN, 1), _f32), jnp.zeros((BN, F_HID - F_OUT - 1), _f32)],
        axis=1)
    acc[...] += lax.dot_general(oh, ext, (((0,), (0,)), ((), ())),
                                preferred_element_type=_f32)

    @pl.when(i == NBLK - 1)
    def _():
        out_ref[...] = acc[:, 0:F_OUT] / jnp.maximum(acc[:, F_OUT:F_OUT + 1], 1.0)


def _fin_call(p2, h, p, w2, b2r, wl, blr, bf):
    blk = pl.BlockSpec((BN, 128), lambda i: (i, 0))
    return pl.pallas_call(
        _fin_body,
        grid=(NBLK,),
        in_specs=[blk, blk, blk,
            pl.BlockSpec((F_HID, F_HID), lambda i: (0, 0)),
            pl.BlockSpec((1, F_HID), lambda i: (0, 0)),
            pl.BlockSpec((F_OUT, F_HID), lambda i: (0, 0)),
            pl.BlockSpec((1, F_OUT), lambda i: (0, 0)),
            pl.BlockSpec((BN, 1), lambda i: (i, 0)),
        ],
        out_specs=pl.BlockSpec((G, F_OUT), lambda i: (0, 0)),
        out_shape=jax.ShapeDtypeStruct((G, F_OUT), _f32),
        scratch_shapes=[pltpu.VMEM((G, F_HID), _f32)],
    )(p2, h, p, w2, b2r, wl, blr, bf)


def kernel(x, edge_index, batch, W1, b1, W2, b2, Wlin, blin):
    src = edge_index[0]
    dst = edge_index[1]
    pad_e = EP - E
    src2 = jnp.concatenate([src, jnp.zeros((pad_e,), jnp.int32)]).reshape(EPR, K)
    dst2 = jnp.concatenate([dst, jnp.full((pad_e,), N, jnp.int32)]).reshape(EPR, K)
    sd = jnp.concatenate(
        [jnp.stack([src2, dst2], axis=1).reshape(2 * EPR, K),
         jnp.zeros((4 * CK, K), jnp.int32)])

    x16 = jnp.zeros((NP, 16), _f32).at[:N, :7].set(x)
    w1p = jnp.zeros((16, F_HID), _f32).at[:7, :].set(W1)
    b1r = b1.reshape(1, F_HID)
    b2r = b2.reshape(1, F_HID)
    blr = blin.reshape(1, F_OUT)
    bf = jnp.full((NP, 1), float(G), _f32).at[:N, 0].set(batch.astype(_f32))

    ones8 = jnp.ones((K, 8), _f32)
    zeros16 = jnp.zeros((ZR, 16), _f32)
    zeros8 = jnp.zeros((ZR, 8), _f32)

    degd = _deg_kernel(sd, ones8, zeros8)
    p = _prep_call(degd, x16)
    o1 = _prop1_kernel(sd, p, zeros16)
    h = _h1_call(o1, p, w1p, b1r)
    p2 = _prop2_kernel(sd, h, zeros16)
    return _fin_call(p2, h, p, W2, b2r, Wlin, blr, bf)
